# Initial kernel scaffold; baseline (speedup 1.0000x reference)
import functools

import jax
import jax.numpy as jnp
from jax import lax
from jax.experimental import pallas as pl
from jax.experimental.pallas import tpu as pltpu
from jax.experimental.pallas import tpu_sc as plsc

NC = 2
NS = 16
NW = NC * NS
EB = 128

_MESH = plsc.VectorSubcoreMesh(core_axis_name="c", subcore_axis_name="s")


def _make_sc_deg(n_pad, nb):
    dz = n_pad // NS

    @functools.partial(
        pl.kernel,
        out_type=jax.ShapeDtypeStruct((NC, n_pad), jnp.float32),
        mesh=_MESH,
        scratch_types=[
            pltpu.VMEM_SHARED((n_pad,), jnp.float32),
            pltpu.VMEM((nb, EB), jnp.int32),
            pltpu.VMEM((EB,), jnp.float32),
        ],
    )
    def deg_kernel(dst_hbm, zeros_hbm, ones_hbm, out_hbm, acc, idx_v, ones_v):
        c = lax.axis_index("c")
        s = lax.axis_index("s")
        wid = c * NS + s
        pltpu.sync_copy(zeros_hbm.at[pl.ds(s * dz, dz)], acc.at[pl.ds(s * dz, dz)])
        pltpu.sync_copy(ones_hbm, ones_v)
        pltpu.sync_copy(dst_hbm.at[wid], idx_v)
        plsc.subcore_barrier()

        def body(j, carry):
            pltpu.sync_copy(ones_v, acc.at[idx_v.at[j]], add=True)
            return carry

        lax.fori_loop(0, nb, body, 0)
        plsc.subcore_barrier()
        pltpu.sync_copy(acc.at[pl.ds(s * dz, dz)], out_hbm.at[c, pl.ds(s * dz, dz)])

    return deg_kernel


def _make_sc_prop(n, r_pad, nb, width):
    rz = r_pad // NS
    rw = n // NS

    @functools.partial(
        pl.kernel,
        out_type=jax.ShapeDtypeStruct((NC, n, width), jnp.float32),
        mesh=_MESH,
        scratch_types=[
            pltpu.VMEM_SHARED((r_pad, width), jnp.float32),
            pltpu.VMEM((nb, EB), jnp.int32),
            pltpu.VMEM((nb, EB), jnp.int32),
            pltpu.VMEM((EB, width), jnp.float32),
            pltpu.SemaphoreType.DMA,
        ],
    )
    def prop_kernel(g_hbm, src_hbm, dst_hbm, zeros_hbm, out_hbm,
                    acc, sidx, didx, rows, sem):
        c = lax.axis_index("c")
        s = lax.axis_index("s")
        wid = c * NS + s
        pltpu.sync_copy(zeros_hbm.at[pl.ds(s * rz, rz)], acc.at[pl.ds(s * rz, rz)])
        pltpu.sync_copy(src_hbm.at[wid], sidx)
        pltpu.sync_copy(dst_hbm.at[wid], didx)
        plsc.subcore_barrier()

        def body(j, carry):
            pltpu.async_copy(g_hbm.at[sidx.at[j]], rows, sem).wait()
            pltpu.sync_copy(rows, acc.at[didx.at[j]], add=True)
            return carry

        lax.fori_loop(0, nb, body, 0)
        plsc.subcore_barrier()
        pltpu.sync_copy(acc.at[pl.ds(s * rw, rw)],
                        out_hbm.at[c, pl.ds(s * rw, rw)])

    return prop_kernel


def _tc_matmul(x, w, bn):
    m, k = x.shape
    n = w.shape[1]

    def body(x_ref, w_ref, o_ref):
        o_ref[...] = jnp.dot(x_ref[...], w_ref[...],
                             preferred_element_type=jnp.float32)

    return pl.pallas_call(
        body,
        grid=(m // bn,),
        in_specs=[
            pl.BlockSpec((bn, k), lambda i: (i, 0)),
            pl.BlockSpec((k, n), lambda i: (0, 0)),
        ],
        out_specs=pl.BlockSpec((bn, n), lambda i: (i, 0)),
        out_shape=jax.ShapeDtypeStruct((m, n), jnp.float32),
    )(x, w)


def _tc_scale(deg2, h0, bn):
    n, w = h0.shape

    def body(deg_ref, h0_ref, dv_ref, g0_ref):
        deg = deg_ref[:, 0:1] + deg_ref[:, 1:2] + 1.0
        dinv = lax.rsqrt(jnp.maximum(deg, 1.0))
        dv = jnp.broadcast_to(dinv, (bn, w))
        dv_ref[...] = dv
        g0_ref[...] = dv * h0_ref[...]

    return pl.pallas_call(
        body,
        grid=(n // bn,),
        in_specs=[
            pl.BlockSpec((bn, 2), lambda i: (i, 0)),
            pl.BlockSpec((bn, w), lambda i: (i, 0)),
        ],
        out_specs=[
            pl.BlockSpec((bn, w), lambda i: (i, 0)),
            pl.BlockSpec((bn, w), lambda i: (i, 0)),
        ],
        out_shape=[
            jax.ShapeDtypeStruct((n, w), jnp.float32),
            jax.ShapeDtypeStruct((n, w), jnp.float32),
        ],
    )(deg2, h0)


def _tc_layer1(s1, g0, dv, b1, bn):
    n, w = g0.shape

    def body(s_ref, g0_ref, dv_ref, b1_ref, h1_ref, g1_ref):
        t = s_ref[0] + s_ref[1] + g0_ref[...]
        h1 = jnp.maximum(dv_ref[...] * t + b1_ref[...], 0.0)
        h1_ref[...] = h1
        g1_ref[...] = dv_ref[...] * h1

    return pl.pallas_call(
        body,
        grid=(n // bn,),
        in_specs=[
            pl.BlockSpec((2, bn, w), lambda i: (0, i, 0)),
            pl.BlockSpec((bn, w), lambda i: (i, 0)),
            pl.BlockSpec((bn, w), lambda i: (i, 0)),
            pl.BlockSpec((1, w), lambda i: (0, 0)),
        ],
        out_specs=[
            pl.BlockSpec((bn, w), lambda i: (i, 0)),
            pl.BlockSpec((bn, w), lambda i: (i, 0)),
        ],
        out_shape=[
            jax.ShapeDtypeStruct((n, w), jnp.float32),
            jax.ShapeDtypeStruct((n, w), jnp.float32),
        ],
    )(s1, g0, dv, b1)


def _tc_layer2(s2, g1, dv, w2, b2, bn):
    n, w = g1.shape
    cdim = w2.shape[1]

    def body(s_ref, g1_ref, dv_ref, w2_ref, b2_ref, o_ref):
        t = dv_ref[...] * (s_ref[0] + s_ref[1] + g1_ref[...])
        z = jnp.dot(t, w2_ref[...], preferred_element_type=jnp.float32)
        z = z + b2_ref[...]
        m = jnp.max(z, axis=1, keepdims=True)
        ez = jnp.exp(z - m)
        lse = jnp.log(jnp.sum(ez, axis=1, keepdims=True)) + m
        o_ref[...] = z - lse

    return pl.pallas_call(
        body,
        grid=(n // bn,),
        in_specs=[
            pl.BlockSpec((2, bn, w), lambda i: (0, i, 0)),
            pl.BlockSpec((bn, w), lambda i: (i, 0)),
            pl.BlockSpec((bn, w), lambda i: (i, 0)),
            pl.BlockSpec((w, cdim), lambda i: (0, 0)),
            pl.BlockSpec((1, cdim), lambda i: (0, 0)),
        ],
        out_specs=pl.BlockSpec((bn, cdim), lambda i: (i, 0)),
        out_shape=jax.ShapeDtypeStruct((n, cdim), jnp.float32),
    )(s2, g1, dv, w2, b2)


def kernel(input_matrix, edge_index, W1, b1, W2, b2):
    n, d = input_matrix.shape
    h = W1.shape[1]
    c = W2.shape[1]
    e = edge_index.shape[1]

    ew = -(-e // (NW * EB)) * EB
    nb = ew // EB
    e_pad = NW * ew
    src = jnp.concatenate([edge_index[0], jnp.zeros((e_pad - e,), jnp.int32)])
    dst = jnp.concatenate([edge_index[1], jnp.full((e_pad - e,), n, jnp.int32)])
    src3 = src.reshape(NW, nb, EB)
    dst3 = dst.reshape(NW, nb, EB)

    deg_pad = -(-(n + 1) // (16 * NS)) * (16 * NS)
    r_pad = -(-(n + 1) // NS) * NS

    zeros_deg = jnp.zeros((deg_pad,), jnp.float32)
    ones_eb = jnp.ones((EB,), jnp.float32)
    zeros_acc = jnp.zeros((r_pad, h), jnp.float32)

    bn = 2000

    deg_part = _make_sc_deg(deg_pad, nb)(dst3, zeros_deg, ones_eb)
    h0 = _tc_matmul(input_matrix, W1, bn)

    deg2 = deg_part[:, :n].T
    dv, g0 = _tc_scale(deg2, h0, bn)

    prop = _make_sc_prop(n, r_pad, nb, h)
    s1 = prop(g0, src3, dst3, zeros_acc)
    h1, g1 = _tc_layer1(s1, g0, dv, b1.reshape(1, h), bn)

    s2 = prop(g1, src3, dst3, zeros_acc)
    return _tc_layer2(s2, g1, dv, W2, b2.reshape(1, c), bn)


# speedup vs baseline: 21.4467x; 21.4467x over previous
import functools

import jax
import jax.numpy as jnp
from jax import lax
from jax.experimental import pallas as pl
from jax.experimental.pallas import tpu as pltpu
from jax.experimental.pallas import tpu_sc as plsc

NC = 2
NS = 16
NW = NC * NS
EB = 128

_MESH = plsc.VectorSubcoreMesh(core_axis_name="c", subcore_axis_name="s")


def _make_sc_deg(n_pad, nb):
    dz = n_pad // NS

    @functools.partial(
        pl.kernel,
        out_type=jax.ShapeDtypeStruct((NC, n_pad), jnp.float32),
        mesh=_MESH,
        scratch_types=[
            pltpu.VMEM_SHARED((n_pad,), jnp.float32),
            pltpu.VMEM((nb, EB), jnp.int32),
            pltpu.VMEM((EB,), jnp.float32),
        ],
    )
    def deg_kernel(dst_hbm, zeros_hbm, ones_hbm, out_hbm, acc, idx_v, ones_v):
        c = lax.axis_index("c")
        s = lax.axis_index("s")
        wid = c * NS + s
        pltpu.sync_copy(zeros_hbm.at[pl.ds(s * dz, dz)], acc.at[pl.ds(s * dz, dz)])
        pltpu.sync_copy(ones_hbm, ones_v)
        pltpu.sync_copy(dst_hbm.at[wid], idx_v)
        plsc.subcore_barrier()

        def body(j, carry):
            pltpu.sync_copy(ones_v, acc.at[idx_v.at[j]], add=True)
            return carry

        lax.fori_loop(0, nb, body, 0)
        plsc.subcore_barrier()
        pltpu.sync_copy(acc.at[pl.ds(s * dz, dz)], out_hbm.at[c, pl.ds(s * dz, dz)])

    return deg_kernel


def _make_sc_prop(r_pad, nb, width):
    rz = r_pad // NS

    @functools.partial(
        pl.kernel,
        out_type=jax.ShapeDtypeStruct((NC, r_pad, width), jnp.float32),
        mesh=_MESH,
        scratch_types=[
            pltpu.VMEM_SHARED((r_pad, width), jnp.float32),
            pltpu.VMEM((nb, EB), jnp.int32),
            pltpu.VMEM((nb, EB), jnp.int32),
            pltpu.VMEM((EB, width), jnp.float32),
            pltpu.SemaphoreType.DMA,
        ],
        compiler_params=pltpu.CompilerParams(use_tc_tiling_on_sc=False),
    )
    def prop_kernel(g_hbm, src_hbm, dst_hbm, zeros_hbm, out_hbm,
                    acc, sidx, didx, rows, sem):
        c = lax.axis_index("c")
        s = lax.axis_index("s")
        wid = c * NS + s
        pltpu.sync_copy(zeros_hbm.at[pl.ds(s * rz, rz)], acc.at[pl.ds(s * rz, rz)])
        pltpu.sync_copy(src_hbm.at[wid], sidx)
        pltpu.sync_copy(dst_hbm.at[wid], didx)
        plsc.subcore_barrier()

        def body(j, carry):
            pltpu.async_copy(g_hbm.at[sidx.at[j]], rows, sem).wait()
            pltpu.sync_copy(rows, acc.at[didx.at[j]], add=True)
            return carry

        lax.fori_loop(0, nb, body, 0)
        plsc.subcore_barrier()
        pltpu.sync_copy(acc.at[pl.ds(s * rz, rz)],
                        out_hbm.at[c, pl.ds(s * rz, rz)])

    return prop_kernel


def _tc_matmul(x, w, bn):
    m, k = x.shape
    n = w.shape[1]

    def body(x_ref, w_ref, o_ref):
        o_ref[...] = jnp.dot(x_ref[...], w_ref[...],
                             preferred_element_type=jnp.float32)

    return pl.pallas_call(
        body,
        grid=(m // bn,),
        in_specs=[
            pl.BlockSpec((bn, k), lambda i: (i, 0)),
            pl.BlockSpec((k, n), lambda i: (0, 0)),
        ],
        out_specs=pl.BlockSpec((bn, n), lambda i: (i, 0)),
        out_shape=jax.ShapeDtypeStruct((m, n), jnp.float32),
    )(x, w)


def _tc_scale(deg2, h0, bn):
    n, w = h0.shape

    def body(deg_ref, h0_ref, dv_ref, g0_ref):
        deg = deg_ref[:, 0:1] + deg_ref[:, 1:2] + 1.0
        dinv = lax.rsqrt(jnp.maximum(deg, 1.0))
        dv = jnp.broadcast_to(dinv, (bn, w))
        dv_ref[...] = dv
        g0_ref[...] = dv * h0_ref[...]

    return pl.pallas_call(
        body,
        grid=(n // bn,),
        in_specs=[
            pl.BlockSpec((bn, 2), lambda i: (i, 0)),
            pl.BlockSpec((bn, w), lambda i: (i, 0)),
        ],
        out_specs=[
            pl.BlockSpec((bn, w), lambda i: (i, 0)),
            pl.BlockSpec((bn, w), lambda i: (i, 0)),
        ],
        out_shape=[
            jax.ShapeDtypeStruct((n, w), jnp.float32),
            jax.ShapeDtypeStruct((n, w), jnp.float32),
        ],
    )(deg2, h0)


def _tc_layer1(s1, g0, dv, b1, bn):
    n, w = g0.shape

    def body(s_ref, g0_ref, dv_ref, b1_ref, h1_ref, g1_ref):
        t = s_ref[0] + s_ref[1] + g0_ref[...]
        h1 = jnp.maximum(dv_ref[...] * t + b1_ref[...], 0.0)
        h1_ref[...] = h1
        g1_ref[...] = dv_ref[...] * h1

    return pl.pallas_call(
        body,
        grid=(n // bn,),
        in_specs=[
            pl.BlockSpec((2, bn, w), lambda i: (0, i, 0)),
            pl.BlockSpec((bn, w), lambda i: (i, 0)),
            pl.BlockSpec((bn, w), lambda i: (i, 0)),
            pl.BlockSpec((1, w), lambda i: (0, 0)),
        ],
        out_specs=[
            pl.BlockSpec((bn, w), lambda i: (i, 0)),
            pl.BlockSpec((bn, w), lambda i: (i, 0)),
        ],
        out_shape=[
            jax.ShapeDtypeStruct((n, w), jnp.float32),
            jax.ShapeDtypeStruct((n, w), jnp.float32),
        ],
    )(s1, g0, dv, b1)


def _tc_layer2(s2, g1, dv, w2, b2, bn):
    n, w = g1.shape
    cdim = w2.shape[1]

    def body(s_ref, g1_ref, dv_ref, w2_ref, b2_ref, o_ref):
        t = dv_ref[...] * (s_ref[0] + s_ref[1] + g1_ref[...])
        z = jnp.dot(t, w2_ref[...], preferred_element_type=jnp.float32)
        z = z + b2_ref[...]
        m = jnp.max(z, axis=1, keepdims=True)
        ez = jnp.exp(z - m)
        lse = jnp.log(jnp.sum(ez, axis=1, keepdims=True)) + m
        o_ref[...] = z - lse

    return pl.pallas_call(
        body,
        grid=(n // bn,),
        in_specs=[
            pl.BlockSpec((2, bn, w), lambda i: (0, i, 0)),
            pl.BlockSpec((bn, w), lambda i: (i, 0)),
            pl.BlockSpec((bn, w), lambda i: (i, 0)),
            pl.BlockSpec((w, cdim), lambda i: (0, 0)),
            pl.BlockSpec((1, cdim), lambda i: (0, 0)),
        ],
        out_specs=pl.BlockSpec((bn, cdim), lambda i: (i, 0)),
        out_shape=jax.ShapeDtypeStruct((n, cdim), jnp.float32),
    )(s2, g1, dv, w2, b2)


def kernel(input_matrix, edge_index, W1, b1, W2, b2):
    n, d = input_matrix.shape
    h = W1.shape[1]
    c = W2.shape[1]
    e = edge_index.shape[1]

    ew = -(-e // (NW * EB)) * EB
    nb = ew // EB
    e_pad = NW * ew
    src = jnp.concatenate([edge_index[0], jnp.zeros((e_pad - e,), jnp.int32)])
    dst = jnp.concatenate([edge_index[1], jnp.full((e_pad - e,), n, jnp.int32)])
    src3 = src.reshape(NW, nb, EB)
    dst3 = dst.reshape(NW, nb, EB)

    deg_pad = -(-(n + 1) // (16 * NS)) * (16 * NS)
    r_pad = -(-(n + 1) // (8 * NS)) * (8 * NS)

    zeros_deg = jnp.zeros((deg_pad,), jnp.float32)
    ones_eb = jnp.ones((EB,), jnp.float32)
    zeros_acc = jnp.zeros((r_pad, h), jnp.float32)

    bn = 2000

    deg_part = _make_sc_deg(deg_pad, nb)(dst3, zeros_deg, ones_eb)
    h0 = _tc_matmul(input_matrix, W1, bn)

    deg2 = deg_part[:, :n].T
    dv, g0 = _tc_scale(deg2, h0, bn)

    prop = _make_sc_prop(r_pad, nb, h)
    s1 = prop(g0, src3, dst3, zeros_acc)[:, :n]
    h1, g1 = _tc_layer1(s1, g0, dv, b1.reshape(1, h), bn)

    s2 = prop(g1, src3, dst3, zeros_acc)[:, :n]
    return _tc_layer2(s2, g1, dv, W2, b2.reshape(1, c), bn)


# speedup vs baseline: 25.7787x; 1.2020x over previous
import functools

import jax
import jax.numpy as jnp
from jax import lax
from jax.experimental import pallas as pl
from jax.experimental.pallas import tpu as pltpu
from jax.experimental.pallas import tpu_sc as plsc

NC = 2
NS = 16
NW = NC * NS
EB = 128


def _mesh():
    return plsc.VectorSubcoreMesh(core_axis_name="c", subcore_axis_name="s")


def _fast_rsqrt(x):
    i = plsc.bitcast(x, jnp.int32)
    i = jnp.full((16,), 0x5F3759DF, jnp.int32) - lax.shift_right_logical(i, 1)
    y = plsc.bitcast(i, jnp.float32)
    half = x * 0.5
    for _ in range(3):
        y = y * (1.5 - half * y * y)
    return y


def _make_sc_degscale(r_pad, nb2):
    dz = r_pad // NS
    rw = r_pad // NW

    @functools.partial(
        pl.kernel,
        out_type=[
            jax.ShapeDtypeStruct((r_pad, 16), jnp.float32),
            jax.ShapeDtypeStruct((r_pad, 16), jnp.float32),
        ],
        mesh=_mesh(),
        scratch_types=[
            pltpu.VMEM_SHARED((r_pad,), jnp.float32),
            pltpu.VMEM((nb2, EB), jnp.int32),
            pltpu.VMEM((EB,), jnp.float32),
            pltpu.VMEM((rw,), jnp.float32),
            pltpu.VMEM((rw, 16), jnp.float32),
            pltpu.VMEM((rw, 16), jnp.float32),
            pltpu.SemaphoreType.DMA,
        ],
        compiler_params=pltpu.CompilerParams(use_tc_tiling_on_sc=False,
                                             needs_layout_passes=False),
    )
    def degscale(dst_hbm, zeros_hbm, ones_hbm, h0_hbm, dv_hbm, g0_hbm,
                 acc, idx_v, ones_v, ybuf, hbuf, obuf, sem):
        c = lax.axis_index("c")
        s = lax.axis_index("s")
        wid = c * NS + s
        pltpu.sync_copy(zeros_hbm.at[pl.ds(s * dz, dz)], acc.at[pl.ds(s * dz, dz)])
        pltpu.sync_copy(ones_hbm, ones_v)
        pltpu.sync_copy(dst_hbm.at[s], idx_v)
        plsc.subcore_barrier()

        def fire(j, carry):
            pltpu.async_copy(ones_v, acc.at[idx_v.at[j]], sem, add=True)
            return carry

        lax.fori_loop(0, nb2, fire, 0)

        def drain(j, carry):
            pltpu.make_async_copy(ones_v, acc.at[idx_v.at[0]], sem).wait()
            return carry

        lax.fori_loop(0, nb2, drain, 0)
        plsc.subcore_barrier()

        pltpu.sync_copy(acc.at[pl.ds(wid * rw, rw)], ybuf)
        pltpu.sync_copy(h0_hbm.at[pl.ds(wid * rw, rw)], hbuf)

        def rsq(i, carry):
            deg = ybuf[pl.ds(i * 16, 16)] + 1.0
            ybuf[pl.ds(i * 16, 16)] = _fast_rsqrt(deg)
            return carry

        lax.fori_loop(0, rw // 16, rsq, 0)

        def rows(k, carry):
            dvrow = plsc.load_gather(ybuf, [jnp.full((16,), k, jnp.int32)])
            obuf[k] = dvrow
            hbuf[k] = dvrow * hbuf[k]
            return carry

        lax.fori_loop(0, rw, rows, 0)
        pltpu.sync_copy(obuf, dv_hbm.at[pl.ds(wid * rw, rw)])
        pltpu.sync_copy(hbuf, g0_hbm.at[pl.ds(wid * rw, rw)])

    return degscale


def _make_sc_prop(r_pad, nb):
    rz = r_pad // NS
    ew = nb * EB

    @functools.partial(
        pl.kernel,
        out_type=jax.ShapeDtypeStruct((NC, r_pad, 16), jnp.float32),
        mesh=_mesh(),
        scratch_types=[
            pltpu.VMEM_SHARED((r_pad, 16), jnp.float32),
            pltpu.VMEM((nb, EB), jnp.int32),
            pltpu.VMEM((nb, EB), jnp.int32),
            pltpu.VMEM((ew, 16), jnp.float32),
            pltpu.SemaphoreType.DMA,
        ],
        compiler_params=pltpu.CompilerParams(use_tc_tiling_on_sc=False),
    )
    def prop_kernel(g_hbm, src_hbm, dst_hbm, zeros_hbm, out_hbm,
                    acc, sidx, didx, rows, sem):
        c = lax.axis_index("c")
        s = lax.axis_index("s")
        wid = c * NS + s
        pltpu.sync_copy(zeros_hbm.at[pl.ds(s * rz, rz)], acc.at[pl.ds(s * rz, rz)])
        pltpu.sync_copy(src_hbm.at[wid], sidx)
        pltpu.sync_copy(dst_hbm.at[wid], didx)
        plsc.subcore_barrier()

        def fire(j, carry):
            pltpu.async_copy(g_hbm.at[sidx.at[j]], rows.at[pl.ds(j * EB, EB)], sem)
            return carry

        lax.fori_loop(0, nb, fire, 0)
        pltpu.make_async_copy(g_hbm.at[pl.ds(0, ew)], rows, sem).wait()

        def scat(j, carry):
            pltpu.sync_copy(rows.at[pl.ds(j * EB, EB)], acc.at[didx.at[j]], add=True)
            return carry

        lax.fori_loop(0, nb, scat, 0)
        plsc.subcore_barrier()
        pltpu.sync_copy(acc.at[pl.ds(s * rz, rz)],
                        out_hbm.at[c, pl.ds(s * rz, rz)])

    return prop_kernel


def _tc_matmul(x, w, r_pad, bn):
    m, k = x.shape
    n = w.shape[1]

    def body(x_ref, w_ref, o_ref):
        o_ref[...] = jnp.dot(x_ref[...], w_ref[...],
                             preferred_element_type=jnp.float32)

    return pl.pallas_call(
        body,
        grid=(m // bn,),
        in_specs=[
            pl.BlockSpec((bn, k), lambda i: (i, 0)),
            pl.BlockSpec((k, n), lambda i: (0, 0)),
        ],
        out_specs=pl.BlockSpec((bn, n), lambda i: (i, 0)),
        out_shape=jax.ShapeDtypeStruct((r_pad, n), jnp.float32),
    )(x, w)


def _tc_layer1(s1, g0, dv, b1):
    r, w = g0.shape

    def body(s_ref, g0_ref, dv_ref, b1_ref, g1_ref):
        t = s_ref[0] + s_ref[1] + g0_ref[...]
        h1 = jnp.maximum(dv_ref[...] * t + b1_ref[...], 0.0)
        g1_ref[...] = dv_ref[...] * h1

    return pl.pallas_call(
        body,
        grid=(1,),
        in_specs=[
            pl.BlockSpec((2, r, w), lambda i: (0, 0, 0)),
            pl.BlockSpec((r, w), lambda i: (0, 0)),
            pl.BlockSpec((r, w), lambda i: (0, 0)),
            pl.BlockSpec((1, w), lambda i: (0, 0)),
        ],
        out_specs=pl.BlockSpec((r, w), lambda i: (0, 0)),
        out_shape=jax.ShapeDtypeStruct((r, w), jnp.float32),
    )(s1, g0, dv, b1)


def _tc_layer2(s2, g1, dv, w2, b2, n, bn):
    w = g1.shape[1]
    cdim = w2.shape[1]

    def body(s_ref, g1_ref, dv_ref, w2_ref, b2_ref, o_ref):
        t = dv_ref[...] * (s_ref[0] + s_ref[1] + g1_ref[...])
        z = jnp.dot(t, w2_ref[...], preferred_element_type=jnp.float32)
        z = z + b2_ref[...]
        m = jnp.max(z, axis=1, keepdims=True)
        ez = jnp.exp(z - m)
        lse = jnp.log(jnp.sum(ez, axis=1, keepdims=True)) + m
        o_ref[...] = z - lse

    return pl.pallas_call(
        body,
        grid=(n // bn,),
        in_specs=[
            pl.BlockSpec((2, bn, w), lambda i: (0, i, 0)),
            pl.BlockSpec((bn, w), lambda i: (i, 0)),
            pl.BlockSpec((bn, w), lambda i: (i, 0)),
            pl.BlockSpec((w, cdim), lambda i: (0, 0)),
            pl.BlockSpec((1, cdim), lambda i: (0, 0)),
        ],
        out_specs=pl.BlockSpec((bn, cdim), lambda i: (i, 0)),
        out_shape=jax.ShapeDtypeStruct((n, cdim), jnp.float32),
    )(s2, g1, dv, w2, b2)


def kernel(input_matrix, edge_index, W1, b1, W2, b2):
    n, d = input_matrix.shape
    h = W1.shape[1]
    c = W2.shape[1]
    e = edge_index.shape[1]

    ew = -(-e // (NW * EB)) * EB
    nb = ew // EB
    e_pad = NW * ew
    src = jnp.concatenate([edge_index[0], jnp.zeros((e_pad - e,), jnp.int32)])
    dst = jnp.concatenate([edge_index[1], jnp.full((e_pad - e,), n, jnp.int32)])
    src3 = src.reshape(NW, nb, EB)
    dst3 = dst.reshape(NW, nb, EB)
    dst2 = dst.reshape(NS, 2 * nb, EB)

    r_pad = -(-(n + 1) // (16 * NS)) * (16 * NS)

    zeros_deg = jnp.zeros((r_pad,), jnp.float32)
    ones_eb = jnp.ones((EB,), jnp.float32)
    zeros_acc = jnp.zeros((r_pad, 16), jnp.float32)

    bn = 2000

    h0 = _tc_matmul(input_matrix, W1, r_pad, bn)
    dv, g0 = _make_sc_degscale(r_pad, 2 * nb)(dst2, zeros_deg, ones_eb, h0)

    prop = _make_sc_prop(r_pad, nb)
    s1 = prop(g0, src3, dst3, zeros_acc)
    g1 = _tc_layer1(s1, g0, dv, b1.reshape(1, h))

    s2 = prop(g1, src3, dst3, zeros_acc)
    return _tc_layer2(s2, g1, dv, W2, b2.reshape(1, c), n, bn)


# speedup vs baseline: 39.4779x; 1.5314x over previous
import functools

import jax
import jax.numpy as jnp
from jax import lax
from jax.experimental import pallas as pl
from jax.experimental.pallas import tpu as pltpu
from jax.experimental.pallas import tpu_sc as plsc

NC = 2
NS = 16
NW = NC * NS

_SC_PARAMS = dict(
    compiler_params=pltpu.CompilerParams(use_tc_tiling_on_sc=False,
                                         needs_layout_passes=False),
)


def _mesh():
    return plsc.VectorSubcoreMesh(core_axis_name="c", subcore_axis_name="s")


def _fast_rsqrt(x):
    i = plsc.bitcast(x, jnp.int32)
    i = jnp.full((16,), 0x5F3759DF, jnp.int32) - lax.shift_right_logical(i, 1)
    y = plsc.bitcast(i, jnp.float32)
    half = x * 0.5
    for _ in range(3):
        y = y * (1.5 - half * y * y)
    return y


def _make_sc_deg(r_pad, nb2, eb):
    dz = r_pad // NS
    rw = r_pad // NW

    @functools.partial(
        pl.kernel,
        out_type=jax.ShapeDtypeStruct((r_pad, 16), jnp.float32),
        mesh=_mesh(),
        scratch_types=[
            pltpu.VMEM_SHARED((r_pad,), jnp.float32),
            pltpu.VMEM((nb2, eb), jnp.int32),
            pltpu.VMEM((eb,), jnp.float32),
            pltpu.VMEM((rw,), jnp.float32),
            pltpu.VMEM((rw, 16), jnp.float32),
            pltpu.SemaphoreType.DMA,
        ],
        **_SC_PARAMS,
    )
    def deg_kernel(dst_hbm, zeros_hbm, ones_hbm, dv_hbm,
                   acc, idx_v, ones_v, ybuf, obuf, sem):
        c = lax.axis_index("c")
        s = lax.axis_index("s")
        wid = c * NS + s
        pltpu.sync_copy(zeros_hbm.at[pl.ds(s * dz, dz)], acc.at[pl.ds(s * dz, dz)])
        pltpu.sync_copy(ones_hbm, ones_v)
        pltpu.sync_copy(dst_hbm.at[s], idx_v)
        plsc.subcore_barrier()

        def fire(j, carry):
            pltpu.async_copy(ones_v, acc.at[idx_v.at[j]], sem, add=True)
            return carry

        lax.fori_loop(0, nb2, fire, 0)

        def drain(j, carry):
            pltpu.make_async_copy(ones_v, acc.at[idx_v.at[0]], sem).wait()
            return carry

        lax.fori_loop(0, nb2, drain, 0)
        plsc.subcore_barrier()

        pltpu.sync_copy(acc.at[pl.ds(wid * rw, rw)], ybuf)

        def rsq(i, carry):
            deg = ybuf[pl.ds(i * 16, 16)] + 1.0
            ybuf[pl.ds(i * 16, 16)] = _fast_rsqrt(deg)
            return carry

        lax.fori_loop(0, rw // 16, rsq, 0)

        def rows(k, carry):
            obuf[k] = plsc.load_gather(ybuf, [jnp.full((16,), k, jnp.int32)])
            return carry

        lax.fori_loop(0, rw, rows, 0)
        pltpu.sync_copy(obuf, dv_hbm.at[pl.ds(wid * rw, rw)])

    return deg_kernel


def _make_sc_prop(r_pad, nb, eb):
    rz = r_pad // NS
    ew = nb * eb
    nh = nb // 2
    eh = nh * eb

    @functools.partial(
        pl.kernel,
        out_type=jax.ShapeDtypeStruct((NC, r_pad, 16), jnp.float32),
        mesh=_mesh(),
        scratch_types=[
            pltpu.VMEM_SHARED((r_pad, 16), jnp.float32),
            pltpu.VMEM((nb, eb), jnp.int32),
            pltpu.VMEM((nb, eb), jnp.int32),
            pltpu.VMEM((ew, 16), jnp.float32),
            pltpu.SemaphoreType.DMA,
            pltpu.SemaphoreType.DMA,
        ],
        **_SC_PARAMS,
    )
    def prop_kernel(g_hbm, src_hbm, dst_hbm, zeros_hbm, out_hbm,
                    acc, sidx, didx, rows, sema, semb):
        c = lax.axis_index("c")
        s = lax.axis_index("s")
        wid = c * NS + s
        pltpu.sync_copy(zeros_hbm.at[pl.ds(s * rz, rz)], acc.at[pl.ds(s * rz, rz)])
        pltpu.sync_copy(src_hbm.at[wid], sidx)
        pltpu.sync_copy(dst_hbm.at[wid], didx)
        plsc.subcore_barrier()

        def gfire(sem, lo):
            def go(j, carry):
                pltpu.async_copy(g_hbm.at[sidx.at[lo + j]],
                                 rows.at[pl.ds((lo + j) * eb, eb)], sem)
                return carry
            lax.fori_loop(0, nh, go, 0)

        def sfire(sem, lo):
            def go(j, carry):
                pltpu.async_copy(rows.at[pl.ds((lo + j) * eb, eb)],
                                 acc.at[didx.at[lo + j]], sem, add=True)
                return carry
            lax.fori_loop(0, nh, go, 0)

        def drain_half(sem):
            pltpu.make_async_copy(g_hbm.at[pl.ds(0, eh)],
                                  rows.at[pl.ds(0, eh)], sem).wait()

        gfire(sema, 0)
        gfire(semb, nh)
        drain_half(sema)
        sfire(sema, 0)
        drain_half(semb)
        sfire(semb, nh)
        drain_half(sema)
        drain_half(semb)
        plsc.subcore_barrier()
        pltpu.sync_copy(acc.at[pl.ds(s * rz, rz)],
                        out_hbm.at[c, pl.ds(s * rz, rz)])

    return prop_kernel


def _make_sc_scale(r_pad):
    rw = r_pad // NW

    @functools.partial(
        pl.kernel,
        out_type=jax.ShapeDtypeStruct((r_pad, 16), jnp.float32),
        mesh=_mesh(),
        scratch_types=[
            pltpu.VMEM((rw, 16), jnp.float32),
            pltpu.VMEM((rw, 16), jnp.float32),
        ],
        **_SC_PARAMS,
    )
    def scale_kernel(dv_hbm, h0_hbm, g0_hbm, dbuf, hbuf):
        c = lax.axis_index("c")
        s = lax.axis_index("s")
        wid = c * NS + s
        pltpu.sync_copy(dv_hbm.at[pl.ds(wid * rw, rw)], dbuf)
        pltpu.sync_copy(h0_hbm.at[pl.ds(wid * rw, rw)], hbuf)

        def rows(k, carry):
            hbuf[k] = dbuf[k] * hbuf[k]
            return carry

        lax.fori_loop(0, rw, rows, 0)
        pltpu.sync_copy(hbuf, g0_hbm.at[pl.ds(wid * rw, rw)])

    return scale_kernel


def _make_sc_layer1(r_pad):
    rw = r_pad // NW

    @functools.partial(
        pl.kernel,
        out_type=jax.ShapeDtypeStruct((r_pad, 16), jnp.float32),
        mesh=_mesh(),
        scratch_types=[
            pltpu.VMEM((rw, 16), jnp.float32),
            pltpu.VMEM((rw, 16), jnp.float32),
            pltpu.VMEM((rw, 16), jnp.float32),
            pltpu.VMEM((rw, 16), jnp.float32),
            pltpu.VMEM((16,), jnp.float32),
        ],
        **_SC_PARAMS,
    )
    def l1_kernel(s1_hbm, g0_hbm, dv_hbm, b1_hbm, g1_hbm,
                  abuf, bbuf, gbuf, dbuf, bias):
        c = lax.axis_index("c")
        s = lax.axis_index("s")
        wid = c * NS + s
        sl = pl.ds(wid * rw, rw)
        pltpu.sync_copy(s1_hbm.at[0, sl], abuf)
        pltpu.sync_copy(s1_hbm.at[1, sl], bbuf)
        pltpu.sync_copy(g0_hbm.at[sl], gbuf)
        pltpu.sync_copy(dv_hbm.at[sl], dbuf)
        pltpu.sync_copy(b1_hbm, bias)
        b1v = bias[...]

        def rows(k, carry):
            t = abuf[k] + bbuf[k] + gbuf[k]
            h1 = jnp.maximum(dbuf[k] * t + b1v, 0.0)
            gbuf[k] = dbuf[k] * h1
            return carry

        lax.fori_loop(0, rw, rows, 0)
        pltpu.sync_copy(gbuf, g1_hbm.at[sl])

    return l1_kernel


def _make_sc_layer2c(r_pad):
    rw = r_pad // NW

    @functools.partial(
        pl.kernel,
        out_type=jax.ShapeDtypeStruct((r_pad, 16), jnp.float32),
        mesh=_mesh(),
        scratch_types=[
            pltpu.VMEM((rw, 16), jnp.float32),
            pltpu.VMEM((rw, 16), jnp.float32),
            pltpu.VMEM((rw, 16), jnp.float32),
            pltpu.VMEM((rw, 16), jnp.float32),
        ],
        **_SC_PARAMS,
    )
    def l2_kernel(s2_hbm, g1_hbm, dv_hbm, t_hbm, abuf, bbuf, gbuf, dbuf):
        c = lax.axis_index("c")
        s = lax.axis_index("s")
        wid = c * NS + s
        sl = pl.ds(wid * rw, rw)
        pltpu.sync_copy(s2_hbm.at[0, sl], abuf)
        pltpu.sync_copy(s2_hbm.at[1, sl], bbuf)
        pltpu.sync_copy(g1_hbm.at[sl], gbuf)
        pltpu.sync_copy(dv_hbm.at[sl], dbuf)

        def rows(k, carry):
            gbuf[k] = dbuf[k] * (abuf[k] + bbuf[k] + gbuf[k])
            return carry

        lax.fori_loop(0, rw, rows, 0)
        pltpu.sync_copy(gbuf, t_hbm.at[sl])

    return l2_kernel


def _tc_matmul(x, w, r_pad, bn):
    m, k = x.shape
    n = w.shape[1]

    def body(x_ref, w_ref, o_ref):
        o_ref[...] = jnp.dot(x_ref[...], w_ref[...],
                             preferred_element_type=jnp.float32)

    return pl.pallas_call(
        body,
        grid=(m // bn,),
        in_specs=[
            pl.BlockSpec((bn, k), lambda i: (i, 0)),
            pl.BlockSpec((k, n), lambda i: (0, 0)),
        ],
        out_specs=pl.BlockSpec((bn, n), lambda i: (i, 0)),
        out_shape=jax.ShapeDtypeStruct((r_pad, n), jnp.float32),
    )(x, w)


def _tc_final(t, w2, b2, n, bn):
    w = t.shape[1]
    cdim = w2.shape[1]

    def body(t_ref, w2_ref, b2_ref, o_ref):
        z = jnp.dot(t_ref[...], w2_ref[...], preferred_element_type=jnp.float32)
        z = z + b2_ref[...]
        m = jnp.max(z, axis=1, keepdims=True)
        ez = jnp.exp(z - m)
        lse = jnp.log(jnp.sum(ez, axis=1, keepdims=True)) + m
        o_ref[...] = z - lse

    return pl.pallas_call(
        body,
        grid=(n // bn,),
        in_specs=[
            pl.BlockSpec((bn, w), lambda i: (i, 0)),
            pl.BlockSpec((w, cdim), lambda i: (0, 0)),
            pl.BlockSpec((1, cdim), lambda i: (0, 0)),
        ],
        out_specs=pl.BlockSpec((bn, cdim), lambda i: (i, 0)),
        out_shape=jax.ShapeDtypeStruct((n, cdim), jnp.float32),
    )(t, w2, b2)


def kernel(input_matrix, edge_index, W1, b1, W2, b2):
    n, d = input_matrix.shape
    h = W1.shape[1]
    c = W2.shape[1]
    e = edge_index.shape[1]

    eb = 100
    assert e % (NW * eb) == 0
    nb = e // (NW * eb)
    src3 = edge_index[0].reshape(NW, nb, eb)
    dst3 = edge_index[1].reshape(NW, nb, eb)
    dst2 = edge_index[1].reshape(NS, NC * nb, eb)

    r_pad = -(-n // (16 * NS)) * (16 * NS)

    zeros_deg = jnp.zeros((r_pad,), jnp.float32)
    ones_eb = jnp.ones((eb,), jnp.float32)
    zeros_acc = jnp.zeros((r_pad, 16), jnp.float32)

    bn = 2000

    dv = _make_sc_deg(r_pad, NC * nb, eb)(dst2, zeros_deg, ones_eb)
    h0 = _tc_matmul(input_matrix, W1, r_pad, bn)
    g0 = _make_sc_scale(r_pad)(dv, h0)

    prop = _make_sc_prop(r_pad, nb, eb)
    s1 = prop(g0, src3, dst3, zeros_acc)
    g1 = _make_sc_layer1(r_pad)(s1, g0, dv, b1)

    s2 = prop(g1, src3, dst3, zeros_acc)
    t = _make_sc_layer2c(r_pad)(s2, g1, dv)
    return _tc_final(t, W2, b2.reshape(1, c), n, bn)


# speedup vs baseline: 44.0278x; 1.1153x over previous
import functools

import jax
import jax.numpy as jnp
from jax import lax
from jax.experimental import pallas as pl
from jax.experimental.pallas import tpu as pltpu
from jax.experimental.pallas import tpu_sc as plsc

NC = 2
NS = 16
NW = NC * NS

_SC_PARAMS = dict(
    compiler_params=pltpu.CompilerParams(use_tc_tiling_on_sc=False,
                                         needs_layout_passes=False),
)


def _mesh():
    return plsc.VectorSubcoreMesh(core_axis_name="c", subcore_axis_name="s")


def _fast_rsqrt(x):
    i = plsc.bitcast(x, jnp.int32)
    i = jnp.full((16,), 0x5F3759DF, jnp.int32) - lax.shift_right_logical(i, 1)
    y = plsc.bitcast(i, jnp.float32)
    half = x * 0.5
    for _ in range(3):
        y = y * (1.5 - half * y * y)
    return y


def _make_sc_deg(r_pad, nb, eb):
    nb2 = NC * nb
    dz = r_pad // NS
    rw = r_pad // NW

    @functools.partial(
        pl.kernel,
        out_type=jax.ShapeDtypeStruct((r_pad, 16), jnp.float32),
        mesh=_mesh(),
        scratch_types=[
            pltpu.VMEM_SHARED((r_pad,), jnp.float32),
            pltpu.VMEM((nb2, eb), jnp.int32),
            pltpu.VMEM((-(-eb // 16) * 16,), jnp.float32),
            pltpu.VMEM((rw,), jnp.float32),
            pltpu.VMEM((rw, 16), jnp.float32),
            pltpu.SemaphoreType.DMA,
        ],
        **_SC_PARAMS,
    )
    def deg_kernel(e4_hbm, dv_hbm, acc, idx_v, ones_v, ybuf, obuf, sem):
        c = lax.axis_index("c")
        s = lax.axis_index("s")
        wid = c * NS + s

        zero16 = jnp.zeros((16,), jnp.float32)

        def zfill(i, carry):
            ybuf[pl.ds(i * 16, 16)] = zero16
            return carry

        lax.fori_loop(0, rw // 16, zfill, 0)

        def zcopy(i, carry):
            pltpu.sync_copy(ybuf, acc.at[pl.ds(s * dz + i * rw, rw)])
            return carry

        lax.fori_loop(0, dz // rw, zcopy, 0)

        one16 = jnp.full((16,), 1.0, jnp.float32)

        def ofill(i, carry):
            ones_v[pl.ds(i * 16, 16)] = one16
            return carry

        lax.fori_loop(0, -(-eb // 16), ofill, 0)
        ones_s = ones_v.at[pl.ds(0, eb)]
        pltpu.sync_copy(e4_hbm.at[1, 2 * s], idx_v.at[pl.ds(0, nb)])
        pltpu.sync_copy(e4_hbm.at[1, 2 * s + 1], idx_v.at[pl.ds(nb, nb)])
        plsc.subcore_barrier()

        def fire(j, carry):
            pltpu.async_copy(ones_s, acc.at[idx_v.at[j]], sem, add=True)
            return carry

        lax.fori_loop(0, nb2, fire, 0)

        def drain(j, carry):
            pltpu.make_async_copy(ones_s, acc.at[idx_v.at[0]], sem).wait()
            return carry

        lax.fori_loop(0, nb2, drain, 0)
        plsc.subcore_barrier()

        pltpu.sync_copy(acc.at[pl.ds(wid * rw, rw)], ybuf)

        def rsq(i, carry):
            sl = pl.ds(i * 16, 16)
            deg = ybuf[sl] + 1.0
            ybuf[sl] = _fast_rsqrt(deg)
            return carry

        lax.fori_loop(0, rw // 16, rsq, 0)

        def rows(k4, carry):
            for u in range(4):
                k = 4 * k4 + u
                obuf[k] = plsc.load_gather(ybuf, [jnp.full((16,), k, jnp.int32)])
            return carry

        lax.fori_loop(0, rw // 4, rows, 0)
        pltpu.sync_copy(obuf, dv_hbm.at[pl.ds(wid * rw, rw)])

    return deg_kernel


def _make_sc_prop(r_pad, nb, eb, n_chunk):
    rz = r_pad // NS
    ew = nb * eb
    cb = nb // n_chunk
    ce = cb * eb

    @functools.partial(
        pl.kernel,
        out_type=jax.ShapeDtypeStruct((NC, r_pad, 16), jnp.float32),
        mesh=_mesh(),
        scratch_types=[
            pltpu.VMEM_SHARED((r_pad, 16), jnp.float32),
            pltpu.VMEM((nb, eb), jnp.int32),
            pltpu.VMEM((nb, eb), jnp.int32),
            pltpu.VMEM((ew, 16), jnp.float32),
        ] + [pltpu.SemaphoreType.DMA] * n_chunk,
        **_SC_PARAMS,
    )
    def prop_kernel(g_hbm, e4_hbm, out_hbm, acc, sidx, didx, rows, *sems):
        c = lax.axis_index("c")
        s = lax.axis_index("s")
        wid = c * NS + s
        zero16 = jnp.zeros((16,), jnp.float32)

        def zero(i, carry):
            for u in range(4):
                rows[4 * i + u] = zero16
            return carry

        lax.fori_loop(0, rz // 4, zero, 0)
        pltpu.sync_copy(rows.at[pl.ds(0, rz)], acc.at[pl.ds(s * rz, rz)])
        pltpu.sync_copy(e4_hbm.at[0, wid], sidx)
        pltpu.sync_copy(e4_hbm.at[1, wid], didx)
        plsc.subcore_barrier()

        def gfire(k):
            def go(j, carry):
                b = k * cb + j
                pltpu.async_copy(g_hbm.at[sidx.at[b]],
                                 rows.at[pl.ds(b * eb, eb)], sems[k])
                return carry
            lax.fori_loop(0, cb, go, 0)

        def sfire(k):
            def go(j, carry):
                b = k * cb + j
                pltpu.async_copy(rows.at[pl.ds(b * eb, eb)],
                                 acc.at[didx.at[b]], sems[k], add=True)
                return carry
            lax.fori_loop(0, cb, go, 0)

        def drain(k):
            pltpu.make_async_copy(g_hbm.at[pl.ds(0, ce)],
                                  rows.at[pl.ds(0, ce)], sems[k]).wait()

        for k in range(n_chunk):
            gfire(k)
        for k in range(n_chunk):
            drain(k)
            sfire(k)
        for k in range(n_chunk):
            drain(k)
        plsc.subcore_barrier()
        pltpu.sync_copy(acc.at[pl.ds(s * rz, rz)],
                        out_hbm.at[c, pl.ds(s * rz, rz)])

    return prop_kernel


def _make_sc_scale(r_pad):
    rw = r_pad // NW

    @functools.partial(
        pl.kernel,
        out_type=jax.ShapeDtypeStruct((r_pad, 16), jnp.float32),
        mesh=_mesh(),
        scratch_types=[
            pltpu.VMEM((rw, 16), jnp.float32),
            pltpu.VMEM((rw, 16), jnp.float32),
        ],
        **_SC_PARAMS,
    )
    def scale_kernel(dv_hbm, h0_hbm, g0_hbm, dbuf, hbuf):
        c = lax.axis_index("c")
        s = lax.axis_index("s")
        wid = c * NS + s
        pltpu.sync_copy(dv_hbm.at[pl.ds(wid * rw, rw)], dbuf)
        pltpu.sync_copy(h0_hbm.at[pl.ds(wid * rw, rw)], hbuf)

        def rows(k4, carry):
            for u in range(4):
                k = 4 * k4 + u
                hbuf[k] = dbuf[k] * hbuf[k]
            return carry

        lax.fori_loop(0, rw // 4, rows, 0)
        pltpu.sync_copy(hbuf, g0_hbm.at[pl.ds(wid * rw, rw)])

    return scale_kernel


def _make_sc_layer1(r_pad):
    rw = r_pad // NW

    @functools.partial(
        pl.kernel,
        out_type=jax.ShapeDtypeStruct((r_pad, 16), jnp.float32),
        mesh=_mesh(),
        scratch_types=[
            pltpu.VMEM((rw, 16), jnp.float32),
            pltpu.VMEM((rw, 16), jnp.float32),
            pltpu.VMEM((rw, 16), jnp.float32),
            pltpu.VMEM((rw, 16), jnp.float32),
            pltpu.VMEM((16,), jnp.float32),
        ],
        **_SC_PARAMS,
    )
    def l1_kernel(s1_hbm, g0_hbm, dv_hbm, b1_hbm, g1_hbm,
                  abuf, bbuf, gbuf, dbuf, bias):
        c = lax.axis_index("c")
        s = lax.axis_index("s")
        wid = c * NS + s
        sl = pl.ds(wid * rw, rw)
        pltpu.sync_copy(s1_hbm.at[0, sl], abuf)
        pltpu.sync_copy(s1_hbm.at[1, sl], bbuf)
        pltpu.sync_copy(g0_hbm.at[sl], gbuf)
        pltpu.sync_copy(dv_hbm.at[sl], dbuf)
        pltpu.sync_copy(b1_hbm, bias)
        b1v = bias[...]

        def rows(k4, carry):
            for u in range(4):
                k = 4 * k4 + u
                t = abuf[k] + bbuf[k] + gbuf[k]
                h1 = jnp.maximum(dbuf[k] * t + b1v, 0.0)
                gbuf[k] = dbuf[k] * h1
            return carry

        lax.fori_loop(0, rw // 4, rows, 0)
        pltpu.sync_copy(gbuf, g1_hbm.at[sl])

    return l1_kernel


def _make_sc_layer2c(r_pad):
    rw = r_pad // NW

    @functools.partial(
        pl.kernel,
        out_type=jax.ShapeDtypeStruct((r_pad, 16), jnp.float32),
        mesh=_mesh(),
        scratch_types=[
            pltpu.VMEM((rw, 16), jnp.float32),
            pltpu.VMEM((rw, 16), jnp.float32),
            pltpu.VMEM((rw, 16), jnp.float32),
            pltpu.VMEM((rw, 16), jnp.float32),
        ],
        **_SC_PARAMS,
    )
    def l2_kernel(s2_hbm, g1_hbm, dv_hbm, t_hbm, abuf, bbuf, gbuf, dbuf):
        c = lax.axis_index("c")
        s = lax.axis_index("s")
        wid = c * NS + s
        sl = pl.ds(wid * rw, rw)
        pltpu.sync_copy(s2_hbm.at[0, sl], abuf)
        pltpu.sync_copy(s2_hbm.at[1, sl], bbuf)
        pltpu.sync_copy(g1_hbm.at[sl], gbuf)
        pltpu.sync_copy(dv_hbm.at[sl], dbuf)

        def rows(k4, carry):
            for u in range(4):
                k = 4 * k4 + u
                gbuf[k] = dbuf[k] * (abuf[k] + bbuf[k] + gbuf[k])
            return carry

        lax.fori_loop(0, rw // 4, rows, 0)
        pltpu.sync_copy(gbuf, t_hbm.at[sl])

    return l2_kernel


def _tc_matmul(x, w, r_pad, bn):
    m, k = x.shape
    n = w.shape[1]

    def body(x_ref, w_ref, o_ref):
        o_ref[...] = jnp.dot(x_ref[...], w_ref[...],
                             preferred_element_type=jnp.float32)

    return pl.pallas_call(
        body,
        grid=(m // bn,),
        in_specs=[
            pl.BlockSpec((bn, k), lambda i: (i, 0)),
            pl.BlockSpec((k, n), lambda i: (0, 0)),
        ],
        out_specs=pl.BlockSpec((bn, n), lambda i: (i, 0)),
        out_shape=jax.ShapeDtypeStruct((r_pad, n), jnp.float32),
    )(x, w)


def _tc_final(t, w2, b2, n, bn):
    w = t.shape[1]
    cdim = w2.shape[1]

    def body(t_ref, w2_ref, b2_ref, o_ref):
        z = jnp.dot(t_ref[...], w2_ref[...], preferred_element_type=jnp.float32)
        z = z + b2_ref[...]
        m = jnp.max(z, axis=1, keepdims=True)
        ez = jnp.exp(z - m)
        lse = jnp.log(jnp.sum(ez, axis=1, keepdims=True)) + m
        o_ref[...] = z - lse

    return pl.pallas_call(
        body,
        grid=(n // bn,),
        in_specs=[
            pl.BlockSpec((bn, w), lambda i: (i, 0)),
            pl.BlockSpec((w, cdim), lambda i: (0, 0)),
            pl.BlockSpec((1, cdim), lambda i: (0, 0)),
        ],
        out_specs=pl.BlockSpec((bn, cdim), lambda i: (i, 0)),
        out_shape=jax.ShapeDtypeStruct((n, cdim), jnp.float32),
    )(t, w2, b2)


def kernel(input_matrix, edge_index, W1, b1, W2, b2):
    n, d = input_matrix.shape
    h = W1.shape[1]
    c = W2.shape[1]
    e = edge_index.shape[1]

    eb = 100
    assert e % (NW * eb) == 0
    nb = e // (NW * eb)
    e4 = edge_index.reshape(2, NW, nb, eb)

    r_pad = -(-n // (16 * NS)) * (16 * NS)

    bn = 2000

    dv = _make_sc_deg(r_pad, nb, eb)(e4)
    h0 = _tc_matmul(input_matrix, W1, r_pad, bn)
    g0 = _make_sc_scale(r_pad)(dv, h0)

    prop = _make_sc_prop(r_pad, nb, eb, 5)
    s1 = prop(g0, e4)
    g1 = _make_sc_layer1(r_pad)(s1, g0, dv, b1)

    s2 = prop(g1, e4)
    t = _make_sc_layer2c(r_pad)(s2, g1, dv)
    return _tc_final(t, W2, b2.reshape(1, c), n, bn)


# speedup vs baseline: 45.2878x; 1.0286x over previous
import functools

import jax
import jax.numpy as jnp
from jax import lax
from jax.experimental import pallas as pl
from jax.experimental.pallas import tpu as pltpu
from jax.experimental.pallas import tpu_sc as plsc

NC = 2
NS = 16
NW = NC * NS

_SC_PARAMS = dict(
    compiler_params=pltpu.CompilerParams(use_tc_tiling_on_sc=False,
                                         needs_layout_passes=False),
)


def _mesh():
    return plsc.VectorSubcoreMesh(core_axis_name="c", subcore_axis_name="s")


def _fast_rsqrt(x):
    i = plsc.bitcast(x, jnp.int32)
    i = jnp.full((16,), 0x5F3759DF, jnp.int32) - lax.shift_right_logical(i, 1)
    y = plsc.bitcast(i, jnp.float32)
    half = x * 0.5
    for _ in range(3):
        y = y * (1.5 - half * y * y)
    return y


def _make_sc_deg(r_pad, nb, eb):
    nb2 = NC * nb
    dz = r_pad // NS
    rw = r_pad // NW

    @functools.partial(
        pl.kernel,
        out_type=jax.ShapeDtypeStruct((r_pad, 16), jnp.float32),
        mesh=_mesh(),
        scratch_types=[
            pltpu.VMEM_SHARED((r_pad,), jnp.float32),
            pltpu.VMEM((nb2, eb), jnp.int32),
            pltpu.VMEM((-(-eb // 16) * 16,), jnp.float32),
            pltpu.VMEM((rw,), jnp.float32),
            pltpu.VMEM((rw, 16), jnp.float32),
            pltpu.SemaphoreType.DMA,
        ],
        **_SC_PARAMS,
    )
    def deg_kernel(e4_hbm, dv_hbm, acc, idx_v, ones_v, ybuf, obuf, sem):
        c = lax.axis_index("c")
        s = lax.axis_index("s")
        wid = c * NS + s

        zero16 = jnp.zeros((16,), jnp.float32)

        def zfill(i, carry):
            ybuf[pl.ds(i * 16, 16)] = zero16
            return carry

        lax.fori_loop(0, rw // 16, zfill, 0)

        def zcopy(i, carry):
            pltpu.sync_copy(ybuf, acc.at[pl.ds(s * dz + i * rw, rw)])
            return carry

        lax.fori_loop(0, dz // rw, zcopy, 0)

        one16 = jnp.full((16,), 1.0, jnp.float32)

        def ofill(i, carry):
            ones_v[pl.ds(i * 16, 16)] = one16
            return carry

        lax.fori_loop(0, -(-eb // 16), ofill, 0)
        ones_s = ones_v.at[pl.ds(0, eb)]
        pltpu.sync_copy(e4_hbm.at[1, 2 * s], idx_v.at[pl.ds(0, nb)])
        pltpu.sync_copy(e4_hbm.at[1, 2 * s + 1], idx_v.at[pl.ds(nb, nb)])
        plsc.subcore_barrier()

        def fire(j, carry):
            pltpu.async_copy(ones_s, acc.at[idx_v.at[j]], sem, add=True)
            return carry

        lax.fori_loop(0, nb2, fire, 0)

        def drain(j, carry):
            pltpu.make_async_copy(ones_s, acc.at[idx_v.at[0]], sem).wait()
            return carry

        lax.fori_loop(0, nb2, drain, 0)
        plsc.subcore_barrier()

        pltpu.sync_copy(acc.at[pl.ds(wid * rw, rw)], ybuf)

        def rsq(i, carry):
            sl = pl.ds(i * 16, 16)
            deg = ybuf[sl] + 1.0
            ybuf[sl] = _fast_rsqrt(deg)
            return carry

        lax.fori_loop(0, rw // 16, rsq, 0)

        def rows(k4, carry):
            for u in range(4):
                k = 4 * k4 + u
                obuf[k] = plsc.load_gather(ybuf, [jnp.full((16,), k, jnp.int32)])
            return carry

        lax.fori_loop(0, rw // 4, rows, 0)
        pltpu.sync_copy(obuf, dv_hbm.at[pl.ds(wid * rw, rw)])

    return deg_kernel


def _make_sc_prop(r_pad, nb, eb, n_chunk):
    rz = r_pad // NS
    ew = nb * eb
    cb = nb // n_chunk
    ce = cb * eb

    @functools.partial(
        pl.kernel,
        out_type=jax.ShapeDtypeStruct((NC, r_pad, 16), jnp.float32),
        mesh=_mesh(),
        scratch_types=[
            pltpu.VMEM_SHARED((r_pad, 16), jnp.float32),
            pltpu.VMEM((nb, eb), jnp.int32),
            pltpu.VMEM((nb, eb), jnp.int32),
            pltpu.VMEM((ew, 16), jnp.float32),
        ] + [pltpu.SemaphoreType.DMA] * n_chunk,
        **_SC_PARAMS,
    )
    def prop_kernel(g_hbm, e4_hbm, out_hbm, acc, sidx, didx, rows, *sems):
        c = lax.axis_index("c")
        s = lax.axis_index("s")
        wid = c * NS + s
        zero16 = jnp.zeros((16,), jnp.float32)

        def zero(i, carry):
            for u in range(4):
                rows[4 * i + u] = zero16
            return carry

        lax.fori_loop(0, rz // 4, zero, 0)
        pltpu.sync_copy(rows.at[pl.ds(0, rz)], acc.at[pl.ds(s * rz, rz)])
        pltpu.sync_copy(e4_hbm.at[0, wid], sidx)
        pltpu.sync_copy(e4_hbm.at[1, wid], didx)
        plsc.subcore_barrier()

        def gfire(k):
            def go(j, carry):
                b = k * cb + j
                pltpu.async_copy(g_hbm.at[sidx.at[b]],
                                 rows.at[pl.ds(b * eb, eb)], sems[k])
                return carry
            lax.fori_loop(0, cb, go, 0)

        def sfire(k):
            def go(j, carry):
                b = k * cb + j
                pltpu.async_copy(rows.at[pl.ds(b * eb, eb)],
                                 acc.at[didx.at[b]], sems[k], add=True)
                return carry
            lax.fori_loop(0, cb, go, 0)

        def drain(k):
            pltpu.make_async_copy(g_hbm.at[pl.ds(0, ce)],
                                  rows.at[pl.ds(0, ce)], sems[k]).wait()

        for k in range(n_chunk):
            gfire(k)
        for k in range(n_chunk):
            drain(k)
            sfire(k)
        for k in range(n_chunk):
            drain(k)
        plsc.subcore_barrier()
        pltpu.sync_copy(acc.at[pl.ds(s * rz, rz)],
                        out_hbm.at[c, pl.ds(s * rz, rz)])

    return prop_kernel


def _make_sc_layer1(r_pad):
    rw = r_pad // NW

    @functools.partial(
        pl.kernel,
        out_type=jax.ShapeDtypeStruct((r_pad, 16), jnp.float32),
        mesh=_mesh(),
        scratch_types=[
            pltpu.VMEM((rw, 16), jnp.float32),
            pltpu.VMEM((rw, 16), jnp.float32),
            pltpu.VMEM((rw, 16), jnp.float32),
            pltpu.VMEM((rw, 16), jnp.float32),
            pltpu.VMEM((16,), jnp.float32),
        ],
        **_SC_PARAMS,
    )
    def l1_kernel(s1_hbm, g0_hbm, dv_hbm, b1_hbm, g1_hbm,
                  abuf, bbuf, gbuf, dbuf, bias):
        c = lax.axis_index("c")
        s = lax.axis_index("s")
        wid = c * NS + s
        sl = pl.ds(wid * rw, rw)
        pltpu.sync_copy(s1_hbm.at[0, sl], abuf)
        pltpu.sync_copy(s1_hbm.at[1, sl], bbuf)
        pltpu.sync_copy(g0_hbm.at[sl], gbuf)
        pltpu.sync_copy(dv_hbm.at[sl], dbuf)
        pltpu.sync_copy(b1_hbm, bias)
        b1v = bias[...]

        def rows(k4, carry):
            for u in range(4):
                k = 4 * k4 + u
                t = abuf[k] + bbuf[k] + gbuf[k]
                h1 = jnp.maximum(dbuf[k] * t + b1v, 0.0)
                gbuf[k] = dbuf[k] * h1
            return carry

        lax.fori_loop(0, rw // 4, rows, 0)
        pltpu.sync_copy(gbuf, g1_hbm.at[sl])

    return l1_kernel


def _make_sc_layer2c(r_pad):
    rw = r_pad // NW

    @functools.partial(
        pl.kernel,
        out_type=jax.ShapeDtypeStruct((r_pad, 16), jnp.float32),
        mesh=_mesh(),
        scratch_types=[
            pltpu.VMEM((rw, 16), jnp.float32),
            pltpu.VMEM((rw, 16), jnp.float32),
            pltpu.VMEM((rw, 16), jnp.float32),
            pltpu.VMEM((rw, 16), jnp.float32),
        ],
        **_SC_PARAMS,
    )
    def l2_kernel(s2_hbm, g1_hbm, dv_hbm, t_hbm, abuf, bbuf, gbuf, dbuf):
        c = lax.axis_index("c")
        s = lax.axis_index("s")
        wid = c * NS + s
        sl = pl.ds(wid * rw, rw)
        pltpu.sync_copy(s2_hbm.at[0, sl], abuf)
        pltpu.sync_copy(s2_hbm.at[1, sl], bbuf)
        pltpu.sync_copy(g1_hbm.at[sl], gbuf)
        pltpu.sync_copy(dv_hbm.at[sl], dbuf)

        def rows(k4, carry):
            for u in range(4):
                k = 4 * k4 + u
                gbuf[k] = dbuf[k] * (abuf[k] + bbuf[k] + gbuf[k])
            return carry

        lax.fori_loop(0, rw // 4, rows, 0)
        pltpu.sync_copy(gbuf, t_hbm.at[sl])

    return l2_kernel


def _tc_matmul(x, w, r_pad, bn):
    m, k = x.shape
    n = w.shape[1]

    def body(x_ref, w_ref, o_ref):
        o_ref[...] = jnp.dot(x_ref[...], w_ref[...],
                             preferred_element_type=jnp.float32)

    return pl.pallas_call(
        body,
        grid=(m // bn,),
        in_specs=[
            pl.BlockSpec((bn, k), lambda i: (i, 0)),
            pl.BlockSpec((k, n), lambda i: (0, 0)),
        ],
        out_specs=pl.BlockSpec((bn, n), lambda i: (i, 0)),
        out_shape=jax.ShapeDtypeStruct((r_pad, n), jnp.float32),
    )(x, w)


def _tc_final(t, w2, b2, n, bn):
    w = t.shape[1]
    cdim = w2.shape[1]

    def body(t_ref, w2_ref, b2_ref, o_ref):
        z = jnp.dot(t_ref[...], w2_ref[...], preferred_element_type=jnp.float32)
        z = z + b2_ref[...]
        m = jnp.max(z, axis=1, keepdims=True)
        ez = jnp.exp(z - m)
        lse = jnp.log(jnp.sum(ez, axis=1, keepdims=True)) + m
        o_ref[...] = z - lse

    return pl.pallas_call(
        body,
        grid=(n // bn,),
        in_specs=[
            pl.BlockSpec((bn, w), lambda i: (i, 0)),
            pl.BlockSpec((w, cdim), lambda i: (0, 0)),
            pl.BlockSpec((1, cdim), lambda i: (0, 0)),
        ],
        out_specs=pl.BlockSpec((bn, cdim), lambda i: (i, 0)),
        out_shape=jax.ShapeDtypeStruct((n, cdim), jnp.float32),
    )(t, w2, b2)


def _make_sc_scale(r_pad):
    rw = r_pad // NW

    @functools.partial(
        pl.kernel,
        out_type=jax.ShapeDtypeStruct((r_pad, 16), jnp.float32),
        mesh=_mesh(),
        scratch_types=[
            pltpu.VMEM((rw, 16), jnp.float32),
            pltpu.VMEM((rw, 16), jnp.float32),
        ],
        **_SC_PARAMS,
    )
    def scale_kernel(dv_hbm, h0_hbm, g0_hbm, dbuf, hbuf):
        c = lax.axis_index("c")
        s = lax.axis_index("s")
        wid = c * NS + s
        pltpu.sync_copy(dv_hbm.at[pl.ds(wid * rw, rw)], dbuf)
        pltpu.sync_copy(h0_hbm.at[pl.ds(wid * rw, rw)], hbuf)

        def rows(k4, carry):
            for u in range(4):
                k = 4 * k4 + u
                hbuf[k] = dbuf[k] * hbuf[k]
            return carry

        lax.fori_loop(0, rw // 4, rows, 0)
        pltpu.sync_copy(hbuf, g0_hbm.at[pl.ds(wid * rw, rw)])

    return scale_kernel


def kernel(input_matrix, edge_index, W1, b1, W2, b2):
    n, d = input_matrix.shape
    h = W1.shape[1]
    c = W2.shape[1]
    e = edge_index.shape[1]

    eb = 100
    assert e % (NW * eb) == 0
    nb = e // (NW * eb)
    e4 = edge_index.reshape(2, NW, nb, eb)

    r_pad = -(-n // (16 * NS)) * (16 * NS)

    bn = 2000

    dv = _make_sc_deg(r_pad, nb, eb)(e4)
    h0 = _tc_matmul(input_matrix, W1, r_pad, bn)
    g0 = _make_sc_scale(r_pad)(dv, h0)

    prop = _make_sc_prop(r_pad, nb, eb, 10)
    s1 = prop(g0, e4)
    g1 = _make_sc_layer1(r_pad)(s1, g0, dv, b1)

    s2 = prop(g1, e4)
    t = _make_sc_layer2c(r_pad)(s2, g1, dv)
    return _tc_final(t, W2, b2.reshape(1, c), n, bn)


# speedup vs baseline: 49.0568x; 1.0832x over previous
import functools

import jax
import jax.numpy as jnp
from jax import lax
from jax.experimental import pallas as pl
from jax.experimental.pallas import tpu as pltpu
from jax.experimental.pallas import tpu_sc as plsc

NC = 2
NS = 16
NW = NC * NS

_SC_PARAMS = dict(
    compiler_params=pltpu.CompilerParams(use_tc_tiling_on_sc=False,
                                         needs_layout_passes=False),
)


def _mesh():
    return plsc.VectorSubcoreMesh(core_axis_name="c", subcore_axis_name="s")


def _fast_rsqrt(x):
    i = plsc.bitcast(x, jnp.int32)
    i = jnp.full((16,), 0x5F3759DF, jnp.int32) - lax.shift_right_logical(i, 1)
    y = plsc.bitcast(i, jnp.float32)
    half = x * 0.5
    for _ in range(3):
        y = y * (1.5 - half * y * y)
    return y


def _make_sc_deg(r_pad, nb, eb):
    nb2 = NC * nb
    dz = r_pad // NS
    rw = r_pad // NW

    @functools.partial(
        pl.kernel,
        out_type=jax.ShapeDtypeStruct((r_pad, 16), jnp.float32),
        mesh=_mesh(),
        scratch_types=[
            pltpu.VMEM_SHARED((r_pad,), jnp.float32),
            pltpu.VMEM((nb2, eb), jnp.int32),
            pltpu.VMEM((-(-eb // 16) * 16,), jnp.float32),
            pltpu.VMEM((rw,), jnp.float32),
            pltpu.VMEM((rw, 16), jnp.float32),
            pltpu.SemaphoreType.DMA,
        ],
        **_SC_PARAMS,
    )
    def deg_kernel(e4_hbm, dv_hbm, acc, idx_v, ones_v, ybuf, obuf, sem):
        c = lax.axis_index("c")
        s = lax.axis_index("s")
        wid = c * NS + s

        zero16 = jnp.zeros((16,), jnp.float32)

        def zfill(i, carry):
            ybuf[pl.ds(i * 16, 16)] = zero16
            return carry

        lax.fori_loop(0, rw // 16, zfill, 0)

        def zcopy(i, carry):
            pltpu.sync_copy(ybuf, acc.at[pl.ds(s * dz + i * rw, rw)])
            return carry

        lax.fori_loop(0, dz // rw, zcopy, 0)

        one16 = jnp.full((16,), 1.0, jnp.float32)

        def ofill(i, carry):
            ones_v[pl.ds(i * 16, 16)] = one16
            return carry

        lax.fori_loop(0, -(-eb // 16), ofill, 0)
        ones_s = ones_v.at[pl.ds(0, eb)]
        pltpu.sync_copy(e4_hbm.at[1, 2 * s], idx_v.at[pl.ds(0, nb)])
        pltpu.sync_copy(e4_hbm.at[1, 2 * s + 1], idx_v.at[pl.ds(nb, nb)])
        plsc.subcore_barrier()

        def fire(j, carry):
            pltpu.async_copy(ones_s, acc.at[idx_v.at[j]], sem, add=True)
            return carry

        lax.fori_loop(0, nb2, fire, 0)

        def drain(j, carry):
            pltpu.make_async_copy(ones_s, acc.at[idx_v.at[0]], sem).wait()
            return carry

        lax.fori_loop(0, nb2, drain, 0)
        plsc.subcore_barrier()

        pltpu.sync_copy(acc.at[pl.ds(wid * rw, rw)], ybuf)

        def rsq(i, carry):
            sl = pl.ds(i * 16, 16)
            deg = ybuf[sl] + 1.0
            ybuf[sl] = _fast_rsqrt(deg)
            return carry

        lax.fori_loop(0, rw // 16, rsq, 0)

        def rows(k4, carry):
            for u in range(4):
                k = 4 * k4 + u
                obuf[k] = plsc.load_gather(ybuf, [jnp.full((16,), k, jnp.int32)])
            return carry

        lax.fori_loop(0, rw // 4, rows, 0)
        pltpu.sync_copy(obuf, dv_hbm.at[pl.ds(wid * rw, rw)])

    return deg_kernel


def _make_sc_prop(r_pad, nb, eb, n_chunk):
    rz = r_pad // NS
    ew = nb * eb
    cb = nb // n_chunk
    ce = cb * eb

    @functools.partial(
        pl.kernel,
        out_type=jax.ShapeDtypeStruct((NC, r_pad, 16), jnp.float32),
        mesh=_mesh(),
        scratch_types=[
            pltpu.VMEM_SHARED((r_pad, 16), jnp.float32),
            pltpu.VMEM((nb, eb), jnp.int32),
            pltpu.VMEM((nb, eb), jnp.int32),
            pltpu.VMEM((ew, 16), jnp.float32),
        ] + [pltpu.SemaphoreType.DMA] * n_chunk,
        **_SC_PARAMS,
    )
    def prop_kernel(g_hbm, e4_hbm, out_hbm, acc, sidx, didx, rows, *sems):
        c = lax.axis_index("c")
        s = lax.axis_index("s")
        wid = c * NS + s
        zero16 = jnp.zeros((16,), jnp.float32)

        def zero(i, carry):
            for u in range(4):
                rows[4 * i + u] = zero16
            return carry

        lax.fori_loop(0, rz // 4, zero, 0)
        pltpu.sync_copy(rows.at[pl.ds(0, rz)], acc.at[pl.ds(s * rz, rz)])
        pltpu.sync_copy(e4_hbm.at[0, wid], sidx)
        pltpu.sync_copy(e4_hbm.at[1, wid], didx)
        plsc.subcore_barrier()

        def gfire(k):
            def go(j, carry):
                b = k * cb + j
                pltpu.async_copy(g_hbm.at[sidx.at[b]],
                                 rows.at[pl.ds(b * eb, eb)], sems[k])
                return carry
            lax.fori_loop(0, cb, go, 0)

        def sfire(k):
            def go(j, carry):
                b = k * cb + j
                pltpu.async_copy(rows.at[pl.ds(b * eb, eb)],
                                 acc.at[didx.at[b]], sems[k], add=True)
                return carry
            lax.fori_loop(0, cb, go, 0)

        def drain(k):
            pltpu.make_async_copy(g_hbm.at[pl.ds(0, ce)],
                                  rows.at[pl.ds(0, ce)], sems[k]).wait()

        for k in range(n_chunk):
            gfire(k)
        for k in range(n_chunk):
            drain(k)
            sfire(k)
        for k in range(n_chunk):
            drain(k)
        plsc.subcore_barrier()
        pltpu.sync_copy(acc.at[pl.ds(s * rz, rz)],
                        out_hbm.at[c, pl.ds(s * rz, rz)])

    return prop_kernel


def _make_sc_layer1(r_pad):
    rw = r_pad // NW

    @functools.partial(
        pl.kernel,
        out_type=jax.ShapeDtypeStruct((r_pad, 16), jnp.float32),
        mesh=_mesh(),
        scratch_types=[
            pltpu.VMEM((rw, 16), jnp.float32),
            pltpu.VMEM((rw, 16), jnp.float32),
            pltpu.VMEM((rw, 16), jnp.float32),
            pltpu.VMEM((rw, 16), jnp.float32),
            pltpu.VMEM((16,), jnp.float32),
        ],
        **_SC_PARAMS,
    )
    def l1_kernel(s1_hbm, g0_hbm, dv_hbm, b1_hbm, g1_hbm,
                  abuf, bbuf, gbuf, dbuf, bias):
        c = lax.axis_index("c")
        s = lax.axis_index("s")
        wid = c * NS + s
        sl = pl.ds(wid * rw, rw)
        pltpu.sync_copy(s1_hbm.at[0, sl], abuf)
        pltpu.sync_copy(s1_hbm.at[1, sl], bbuf)
        pltpu.sync_copy(g0_hbm.at[sl], gbuf)
        pltpu.sync_copy(dv_hbm.at[sl], dbuf)
        pltpu.sync_copy(b1_hbm, bias)
        b1v = bias[...]

        def rows(k4, carry):
            for u in range(4):
                k = 4 * k4 + u
                t = abuf[k] + bbuf[k] + gbuf[k]
                h1 = jnp.maximum(dbuf[k] * t + b1v, 0.0)
                gbuf[k] = dbuf[k] * h1
            return carry

        lax.fori_loop(0, rw // 4, rows, 0)
        pltpu.sync_copy(gbuf, g1_hbm.at[sl])

    return l1_kernel


def _make_sc_layer2c(r_pad):
    rw = r_pad // NW

    @functools.partial(
        pl.kernel,
        out_type=jax.ShapeDtypeStruct((r_pad, 128), jnp.float32),
        mesh=_mesh(),
        scratch_types=[
            pltpu.VMEM((rw, 16), jnp.float32),
            pltpu.VMEM((rw, 16), jnp.float32),
            pltpu.VMEM((rw, 16), jnp.float32),
            pltpu.VMEM((rw, 16), jnp.float32),
        ],
        **_SC_PARAMS,
    )
    def l2_kernel(s2_hbm, g1_hbm, dv_hbm, t_hbm, abuf, bbuf, gbuf, dbuf):
        c = lax.axis_index("c")
        s = lax.axis_index("s")
        wid = c * NS + s
        sl = pl.ds(wid * rw, rw)
        pltpu.sync_copy(s2_hbm.at[0, sl], abuf)
        pltpu.sync_copy(s2_hbm.at[1, sl], bbuf)
        pltpu.sync_copy(g1_hbm.at[sl], gbuf)
        pltpu.sync_copy(dv_hbm.at[sl], dbuf)

        def rows(k4, carry):
            for u in range(4):
                k = 4 * k4 + u
                gbuf[k] = dbuf[k] * (abuf[k] + bbuf[k] + gbuf[k])
            return carry

        lax.fori_loop(0, rw // 4, rows, 0)
        pltpu.sync_copy(gbuf, t_hbm.at[sl, pl.ds(0, 16)])

    return l2_kernel


def _tc_matmul(x, wp, r_pad, bn):
    m, k = x.shape

    def body(x_ref, w_ref, o_ref):
        o_ref[...] = jnp.dot(x_ref[...], w_ref[...],
                             preferred_element_type=jnp.float32)

    return pl.pallas_call(
        body,
        grid=(m // bn,),
        in_specs=[
            pl.BlockSpec((bn, k), lambda i: (i, 0)),
            pl.BlockSpec((k, 128), lambda i: (0, 0)),
        ],
        out_specs=pl.BlockSpec((bn, 128), lambda i: (i, 0)),
        out_shape=jax.ShapeDtypeStruct((r_pad, 128), jnp.float32),
    )(x, wp)


def _tc_final(t, w2, b2, n, bn):
    cdim = w2.shape[1]

    def body(t_ref, w2_ref, b2_ref, o_ref):
        t16 = t_ref[:, :16]
        z = jnp.dot(t16, w2_ref[...], preferred_element_type=jnp.float32)
        z = z + b2_ref[...]
        m = jnp.max(z, axis=1, keepdims=True)
        ez = jnp.exp(z - m)
        lse = jnp.log(jnp.sum(ez, axis=1, keepdims=True)) + m
        o_ref[...] = z - lse

    return pl.pallas_call(
        body,
        grid=(n // bn,),
        in_specs=[
            pl.BlockSpec((bn, 128), lambda i: (i, 0)),
            pl.BlockSpec((16, cdim), lambda i: (0, 0)),
            pl.BlockSpec((1, cdim), lambda i: (0, 0)),
        ],
        out_specs=pl.BlockSpec((bn, cdim), lambda i: (i, 0)),
        out_shape=jax.ShapeDtypeStruct((n, cdim), jnp.float32),
    )(t, w2, b2)


def _make_sc_scale(r_pad):
    rw = r_pad // NW

    @functools.partial(
        pl.kernel,
        out_type=jax.ShapeDtypeStruct((r_pad, 16), jnp.float32),
        mesh=_mesh(),
        scratch_types=[
            pltpu.VMEM((rw, 16), jnp.float32),
            pltpu.VMEM((rw, 16), jnp.float32),
        ],
        **_SC_PARAMS,
    )
    def scale_kernel(dv_hbm, h0_hbm, g0_hbm, dbuf, hbuf):
        c = lax.axis_index("c")
        s = lax.axis_index("s")
        wid = c * NS + s
        pltpu.sync_copy(dv_hbm.at[pl.ds(wid * rw, rw)], dbuf)
        pltpu.sync_copy(h0_hbm.at[pl.ds(wid * rw, rw), pl.ds(0, 16)], hbuf)

        def rows(k4, carry):
            for u in range(4):
                k = 4 * k4 + u
                hbuf[k] = dbuf[k] * hbuf[k]
            return carry

        lax.fori_loop(0, rw // 4, rows, 0)
        pltpu.sync_copy(hbuf, g0_hbm.at[pl.ds(wid * rw, rw)])

    return scale_kernel


def kernel(input_matrix, edge_index, W1, b1, W2, b2):
    n, d = input_matrix.shape
    h = W1.shape[1]
    c = W2.shape[1]
    e = edge_index.shape[1]

    eb = 100
    assert e % (NW * eb) == 0
    nb = e // (NW * eb)
    e4 = edge_index.reshape(2, NW, nb, eb)

    r_pad = -(-n // (16 * NS)) * (16 * NS)

    bn = 2000

    dv = _make_sc_deg(r_pad, nb, eb)(e4)
    w1p = jnp.pad(W1, ((0, 0), (0, 128 - h)))
    h0 = _tc_matmul(input_matrix, w1p, r_pad, bn)
    g0 = _make_sc_scale(r_pad)(dv, h0)

    prop = _make_sc_prop(r_pad, nb, eb, 10)
    s1 = prop(g0, e4)
    g1 = _make_sc_layer1(r_pad)(s1, g0, dv, b1)

    s2 = prop(g1, e4)
    t = _make_sc_layer2c(r_pad)(s2, g1, dv)
    return _tc_final(t, W2, b2.reshape(1, c), n, bn)


# speedup vs baseline: 50.6513x; 1.0325x over previous
import functools

import jax
import jax.numpy as jnp
from jax import lax
from jax.experimental import pallas as pl
from jax.experimental.pallas import tpu as pltpu
from jax.experimental.pallas import tpu_sc as plsc

NC = 2
NS = 16
NW = NC * NS

_SC_PARAMS = dict(
    compiler_params=pltpu.CompilerParams(use_tc_tiling_on_sc=False,
                                         needs_layout_passes=False),
)


def _mesh():
    return plsc.VectorSubcoreMesh(core_axis_name="c", subcore_axis_name="s")


def _fast_rsqrt(x):
    i = plsc.bitcast(x, jnp.int32)
    i = jnp.full((16,), 0x5F3759DF, jnp.int32) - lax.shift_right_logical(i, 1)
    y = plsc.bitcast(i, jnp.float32)
    half = x * 0.5
    for _ in range(3):
        y = y * (1.5 - half * y * y)
    return y


def _make_sc_deg(r_pad, ew):
    dz = r_pad // NS
    rw = r_pad // NW
    eb = 104
    nb = ew // eb
    tail = ew - nb * eb
    assert tail % 16 == 0

    @functools.partial(
        pl.kernel,
        out_type=jax.ShapeDtypeStruct((r_pad, 16), jnp.float32),
        mesh=_mesh(),
        scratch_types=[
            pltpu.VMEM_SHARED((r_pad,), jnp.float32),
            pltpu.VMEM((ew,), jnp.int32),
            pltpu.VMEM((-(-eb // 16) * 16,), jnp.float32),
            pltpu.VMEM((rw,), jnp.float32),
            pltpu.VMEM((rw, 16), jnp.float32),
            pltpu.SemaphoreType.DMA,
        ],
        **_SC_PARAMS,
    )
    def deg_kernel(e4_hbm, dv_hbm, acc, idx_v, ones_v, ybuf, obuf, sem):
        c = lax.axis_index("c")
        s = lax.axis_index("s")
        wid = c * NS + s

        zero16 = jnp.zeros((16,), jnp.float32)

        def zfill(i, carry):
            ybuf[pl.ds(i * 16, 16)] = zero16
            return carry

        lax.fori_loop(0, rw // 16, zfill, 0)

        def zcopy(i, carry):
            pltpu.sync_copy(ybuf, acc.at[pl.ds(s * dz + i * rw, rw)])
            return carry

        lax.fori_loop(0, dz // rw, zcopy, 0)

        one16 = jnp.full((16,), 1.0, jnp.float32)

        def ofill(i, carry):
            ones_v[pl.ds(i * 16, 16)] = one16
            return carry

        lax.fori_loop(0, -(-eb // 16), ofill, 0)
        ones_s = ones_v.at[pl.ds(0, eb)]
        ones_t = ones_v.at[pl.ds(0, tail)]
        pltpu.sync_copy(e4_hbm.at[1, pl.ds(s * ew, ew)], idx_v)
        plsc.subcore_barrier()

        def fire(j, carry):
            pltpu.async_copy(ones_s, acc.at[idx_v.at[pl.ds(j * eb, eb)]],
                             sem, add=True)
            return carry

        lax.fori_loop(0, nb, fire, 0)
        if tail:
            pltpu.async_copy(ones_t, acc.at[idx_v.at[pl.ds(nb * eb, tail)]],
                             sem, add=True)

        def drain(j, carry):
            pltpu.make_async_copy(ones_s, acc.at[idx_v.at[pl.ds(0, eb)]],
                                  sem).wait()
            return carry

        lax.fori_loop(0, nb, drain, 0)
        if tail:
            pltpu.make_async_copy(ones_t, acc.at[idx_v.at[pl.ds(0, tail)]],
                                  sem).wait()
        plsc.subcore_barrier()

        pltpu.sync_copy(acc.at[pl.ds(wid * rw, rw)], ybuf)

        def rsq(i, carry):
            sl = pl.ds(i * 16, 16)
            deg = ybuf[sl] + 1.0
            ybuf[sl] = _fast_rsqrt(deg)
            return carry

        lax.fori_loop(0, rw // 16, rsq, 0)

        def rows(k4, carry):
            for u in range(4):
                k = 4 * k4 + u
                obuf[k] = plsc.load_gather(ybuf, [jnp.full((16,), k, jnp.int32)])
            return carry

        lax.fori_loop(0, rw // 4, rows, 0)
        pltpu.sync_copy(obuf, dv_hbm.at[pl.ds(wid * rw, rw)])

    return deg_kernel


def _make_sc_prop(r_pad, ew, n_chunk):
    rz = r_pad // NS
    eb = 104
    nb = ew // eb
    tail = ew - nb * eb
    cb = nb // n_chunk
    assert nb % n_chunk == 0 and tail % 8 == 0

    @functools.partial(
        pl.kernel,
        out_type=jax.ShapeDtypeStruct((NC, r_pad, 16), jnp.float32),
        mesh=_mesh(),
        scratch_types=[
            pltpu.VMEM_SHARED((r_pad, 16), jnp.float32),
            pltpu.VMEM((ew,), jnp.int32),
            pltpu.VMEM((ew,), jnp.int32),
            pltpu.VMEM((ew, 16), jnp.float32),
        ] + [pltpu.SemaphoreType.DMA] * n_chunk,
        **_SC_PARAMS,
    )
    def prop_kernel(g_hbm, e4_hbm, out_hbm, acc, sidx, didx, rows, *sems):
        c = lax.axis_index("c")
        s = lax.axis_index("s")
        wid = c * NS + s
        zero16 = jnp.zeros((16,), jnp.float32)

        def zero(i, carry):
            for u in range(4):
                rows[4 * i + u] = zero16
            return carry

        lax.fori_loop(0, rz // 4, zero, 0)
        pltpu.sync_copy(rows.at[pl.ds(0, rz)], acc.at[pl.ds(s * rz, rz)])
        pltpu.sync_copy(e4_hbm.at[0, pl.ds(wid * ew, ew)], sidx)
        pltpu.sync_copy(e4_hbm.at[1, pl.ds(wid * ew, ew)], didx)
        plsc.subcore_barrier()

        def gfire(k):
            def go(j, carry):
                b = k * cb + j
                pltpu.async_copy(g_hbm.at[sidx.at[pl.ds(b * eb, eb)]],
                                 rows.at[pl.ds(b * eb, eb)], sems[k])
                return carry
            lax.fori_loop(0, cb, go, 0)
            if k == n_chunk - 1 and tail:
                pltpu.async_copy(g_hbm.at[sidx.at[pl.ds(nb * eb, tail)]],
                                 rows.at[pl.ds(nb * eb, tail)], sems[k])

        def sfire(k):
            def go(j, carry):
                b = k * cb + j
                pltpu.async_copy(rows.at[pl.ds(b * eb, eb)],
                                 acc.at[didx.at[pl.ds(b * eb, eb)]],
                                 sems[k], add=True)
                return carry
            lax.fori_loop(0, cb, go, 0)
            if k == n_chunk - 1 and tail:
                pltpu.async_copy(rows.at[pl.ds(nb * eb, tail)],
                                 acc.at[didx.at[pl.ds(nb * eb, tail)]],
                                 sems[k], add=True)

        def drain(k):
            ce = cb * eb + (tail if k == n_chunk - 1 else 0)
            pltpu.make_async_copy(g_hbm.at[pl.ds(0, ce)],
                                  rows.at[pl.ds(0, ce)], sems[k]).wait()

        for k in range(n_chunk):
            gfire(k)
        for k in range(n_chunk):
            drain(k)
            sfire(k)
        for k in range(n_chunk):
            drain(k)
        plsc.subcore_barrier()
        pltpu.sync_copy(acc.at[pl.ds(s * rz, rz)],
                        out_hbm.at[c, pl.ds(s * rz, rz)])

    return prop_kernel


def _make_sc_layer1(r_pad):
    rw = r_pad // NW

    @functools.partial(
        pl.kernel,
        out_type=jax.ShapeDtypeStruct((r_pad, 16), jnp.float32),
        mesh=_mesh(),
        scratch_types=[
            pltpu.VMEM((rw, 16), jnp.float32),
            pltpu.VMEM((rw, 16), jnp.float32),
            pltpu.VMEM((rw, 16), jnp.float32),
            pltpu.VMEM((rw, 16), jnp.float32),
            pltpu.VMEM((16,), jnp.float32),
        ],
        **_SC_PARAMS,
    )
    def l1_kernel(s1_hbm, g0_hbm, dv_hbm, b1_hbm, g1_hbm,
                  abuf, bbuf, gbuf, dbuf, bias):
        c = lax.axis_index("c")
        s = lax.axis_index("s")
        wid = c * NS + s
        sl = pl.ds(wid * rw, rw)
        pltpu.sync_copy(s1_hbm.at[0, sl], abuf)
        pltpu.sync_copy(s1_hbm.at[1, sl], bbuf)
        pltpu.sync_copy(g0_hbm.at[sl], gbuf)
        pltpu.sync_copy(dv_hbm.at[sl], dbuf)
        pltpu.sync_copy(b1_hbm, bias)
        b1v = bias[...]

        def rows(k4, carry):
            for u in range(4):
                k = 4 * k4 + u
                t = abuf[k] + bbuf[k] + gbuf[k]
                h1 = jnp.maximum(dbuf[k] * t + b1v, 0.0)
                gbuf[k] = dbuf[k] * h1
            return carry

        lax.fori_loop(0, rw // 4, rows, 0)
        pltpu.sync_copy(gbuf, g1_hbm.at[sl])

    return l1_kernel


def _make_sc_layer2c(r_pad):
    rw = r_pad // NW

    @functools.partial(
        pl.kernel,
        out_type=jax.ShapeDtypeStruct((r_pad, 128), jnp.float32),
        mesh=_mesh(),
        scratch_types=[
            pltpu.VMEM((rw, 16), jnp.float32),
            pltpu.VMEM((rw, 16), jnp.float32),
            pltpu.VMEM((rw, 16), jnp.float32),
            pltpu.VMEM((rw, 16), jnp.float32),
        ],
        **_SC_PARAMS,
    )
    def l2_kernel(s2_hbm, g1_hbm, dv_hbm, t_hbm, abuf, bbuf, gbuf, dbuf):
        c = lax.axis_index("c")
        s = lax.axis_index("s")
        wid = c * NS + s
        sl = pl.ds(wid * rw, rw)
        pltpu.sync_copy(s2_hbm.at[0, sl], abuf)
        pltpu.sync_copy(s2_hbm.at[1, sl], bbuf)
        pltpu.sync_copy(g1_hbm.at[sl], gbuf)
        pltpu.sync_copy(dv_hbm.at[sl], dbuf)

        def rows(k4, carry):
            for u in range(4):
                k = 4 * k4 + u
                gbuf[k] = dbuf[k] * (abuf[k] + bbuf[k] + gbuf[k])
            return carry

        lax.fori_loop(0, rw // 4, rows, 0)
        pltpu.sync_copy(gbuf, t_hbm.at[sl, pl.ds(0, 16)])

    return l2_kernel


def _tc_matmul(x, wp, r_pad, bn):
    m, k = x.shape

    def body(x_ref, w_ref, o_ref):
        o_ref[...] = jnp.dot(x_ref[...], w_ref[...],
                             preferred_element_type=jnp.float32)

    return pl.pallas_call(
        body,
        grid=(m // bn,),
        in_specs=[
            pl.BlockSpec((bn, k), lambda i: (i, 0)),
            pl.BlockSpec((k, 128), lambda i: (0, 0)),
        ],
        out_specs=pl.BlockSpec((bn, 128), lambda i: (i, 0)),
        out_shape=jax.ShapeDtypeStruct((r_pad, 128), jnp.float32),
    )(x, wp)


def _tc_final(t, w2, b2, n, bn):
    cdim = w2.shape[1]

    def body(t_ref, w2_ref, b2_ref, o_ref):
        t16 = t_ref[:, :16]
        z = jnp.dot(t16, w2_ref[...], preferred_element_type=jnp.float32)
        z = z + b2_ref[...]
        m = jnp.max(z, axis=1, keepdims=True)
        ez = jnp.exp(z - m)
        lse = jnp.log(jnp.sum(ez, axis=1, keepdims=True)) + m
        o_ref[...] = z - lse

    return pl.pallas_call(
        body,
        grid=(n // bn,),
        in_specs=[
            pl.BlockSpec((bn, 128), lambda i: (i, 0)),
            pl.BlockSpec((16, cdim), lambda i: (0, 0)),
            pl.BlockSpec((1, cdim), lambda i: (0, 0)),
        ],
        out_specs=pl.BlockSpec((bn, cdim), lambda i: (i, 0)),
        out_shape=jax.ShapeDtypeStruct((n, cdim), jnp.float32),
    )(t, w2, b2)


def _make_sc_scale(r_pad):
    rw = r_pad // NW

    @functools.partial(
        pl.kernel,
        out_type=jax.ShapeDtypeStruct((r_pad, 16), jnp.float32),
        mesh=_mesh(),
        scratch_types=[
            pltpu.VMEM((rw, 16), jnp.float32),
            pltpu.VMEM((rw, 16), jnp.float32),
        ],
        **_SC_PARAMS,
    )
    def scale_kernel(dv_hbm, h0_hbm, g0_hbm, dbuf, hbuf):
        c = lax.axis_index("c")
        s = lax.axis_index("s")
        wid = c * NS + s
        pltpu.sync_copy(dv_hbm.at[pl.ds(wid * rw, rw)], dbuf)
        pltpu.sync_copy(h0_hbm.at[pl.ds(wid * rw, rw), pl.ds(0, 16)], hbuf)

        def rows(k4, carry):
            for u in range(4):
                k = 4 * k4 + u
                hbuf[k] = dbuf[k] * hbuf[k]
            return carry

        lax.fori_loop(0, rw // 4, rows, 0)
        pltpu.sync_copy(hbuf, g0_hbm.at[pl.ds(wid * rw, rw)])

    return scale_kernel


def kernel(input_matrix, edge_index, W1, b1, W2, b2):
    n, d = input_matrix.shape
    h = W1.shape[1]
    c = W2.shape[1]
    e = edge_index.shape[1]

    assert e % (NW * 8) == 0

    r_pad = -(-n // (16 * NS)) * (16 * NS)

    bn = 2000

    dv = _make_sc_deg(r_pad, e // NS)(edge_index)
    w1p = jnp.pad(W1, ((0, 0), (0, 128 - h)))
    h0 = _tc_matmul(input_matrix, w1p, r_pad, bn)
    g0 = _make_sc_scale(r_pad)(dv, h0)

    prop = _make_sc_prop(r_pad, e // NW, 8)
    s1 = prop(g0, edge_index)
    g1 = _make_sc_layer1(r_pad)(s1, g0, dv, b1)

    s2 = prop(g1, edge_index)
    t = _make_sc_layer2c(r_pad)(s2, g1, dv)
    return _tc_final(t, W2, b2.reshape(1, c), n, bn)


# speedup vs baseline: 53.0045x; 1.0465x over previous
import functools

import jax
import jax.numpy as jnp
from jax import lax
from jax.experimental import pallas as pl
from jax.experimental.pallas import tpu as pltpu
from jax.experimental.pallas import tpu_sc as plsc

NC = 2
NS = 16
NW = NC * NS

_SC_PARAMS = dict(
    compiler_params=pltpu.CompilerParams(use_tc_tiling_on_sc=False,
                                         needs_layout_passes=False),
)


def _mesh():
    return plsc.VectorSubcoreMesh(core_axis_name="c", subcore_axis_name="s")


def _fast_rsqrt(x):
    i = plsc.bitcast(x, jnp.int32)
    i = jnp.full((16,), 0x5F3759DF, jnp.int32) - lax.shift_right_logical(i, 1)
    y = plsc.bitcast(i, jnp.float32)
    half = x * 0.5
    for _ in range(3):
        y = y * (1.5 - half * y * y)
    return y


def _make_sc_deg(r_pad, ew):
    dz = r_pad // NS
    rw = r_pad // NW
    eb = 104
    nb = ew // eb
    tail = ew - nb * eb
    assert tail % 16 == 0

    @functools.partial(
        pl.kernel,
        out_type=jax.ShapeDtypeStruct((r_pad, 16), jnp.float32),
        mesh=_mesh(),
        scratch_types=[
            pltpu.VMEM_SHARED((r_pad,), jnp.float32),
            pltpu.VMEM((ew,), jnp.int32),
            pltpu.VMEM((-(-eb // 16) * 16,), jnp.float32),
            pltpu.VMEM((rw,), jnp.float32),
            pltpu.VMEM((rw, 16), jnp.float32),
            pltpu.SemaphoreType.DMA,
        ],
        **_SC_PARAMS,
    )
    def deg_kernel(e4_hbm, dv_hbm, acc, idx_v, ones_v, ybuf, obuf, sem):
        c = lax.axis_index("c")
        s = lax.axis_index("s")
        wid = c * NS + s

        zero16 = jnp.zeros((16,), jnp.float32)

        def zfill(i, carry):
            ybuf[pl.ds(i * 16, 16)] = zero16
            return carry

        lax.fori_loop(0, rw // 16, zfill, 0)

        def zcopy(i, carry):
            pltpu.sync_copy(ybuf, acc.at[pl.ds(s * dz + i * rw, rw)])
            return carry

        lax.fori_loop(0, dz // rw, zcopy, 0)

        one16 = jnp.full((16,), 1.0, jnp.float32)

        def ofill(i, carry):
            ones_v[pl.ds(i * 16, 16)] = one16
            return carry

        lax.fori_loop(0, -(-eb // 16), ofill, 0)
        ones_s = ones_v.at[pl.ds(0, eb)]
        ones_t = ones_v.at[pl.ds(0, tail)]
        pltpu.sync_copy(e4_hbm.at[1, pl.ds(s * ew, ew)], idx_v)
        plsc.subcore_barrier()

        def fire(j, carry):
            pltpu.async_copy(ones_s, acc.at[idx_v.at[pl.ds(j * eb, eb)]],
                             sem, add=True)
            return carry

        lax.fori_loop(0, nb, fire, 0)
        if tail:
            pltpu.async_copy(ones_t, acc.at[idx_v.at[pl.ds(nb * eb, tail)]],
                             sem, add=True)

        def drain(j, carry):
            pltpu.make_async_copy(ones_s, acc.at[idx_v.at[pl.ds(0, eb)]],
                                  sem).wait()
            return carry

        lax.fori_loop(0, nb, drain, 0)
        if tail:
            pltpu.make_async_copy(ones_t, acc.at[idx_v.at[pl.ds(0, tail)]],
                                  sem).wait()
        plsc.subcore_barrier()

        pltpu.sync_copy(acc.at[pl.ds(wid * rw, rw)], ybuf)

        def rsq(i, carry):
            sl = pl.ds(i * 16, 16)
            deg = ybuf[sl] + 1.0
            ybuf[sl] = _fast_rsqrt(deg)
            return carry

        lax.fori_loop(0, rw // 16, rsq, 0)

        def rows(k4, carry):
            for u in range(4):
                k = 4 * k4 + u
                obuf[k] = plsc.load_gather(ybuf, [jnp.full((16,), k, jnp.int32)])
            return carry

        lax.fori_loop(0, rw // 4, rows, 0)
        pltpu.sync_copy(obuf, dv_hbm.at[pl.ds(wid * rw, rw)])

    return deg_kernel


def _make_sc_prop(r_pad, ew, n_chunk):
    rz = r_pad // NS
    eb = 104
    nb = ew // eb
    tail = ew - nb * eb
    cb = nb // n_chunk
    assert nb % n_chunk == 0 and tail % 8 == 0

    @functools.partial(
        pl.kernel,
        out_type=jax.ShapeDtypeStruct((NC, r_pad, 16), jnp.float32),
        mesh=_mesh(),
        scratch_types=[
            pltpu.VMEM_SHARED((r_pad, 16), jnp.float32),
            pltpu.VMEM((ew,), jnp.int32),
            pltpu.VMEM((ew,), jnp.int32),
            pltpu.VMEM((ew, 16), jnp.float32),
        ] + [pltpu.SemaphoreType.DMA] * n_chunk,
        **_SC_PARAMS,
    )
    def prop_kernel(g_hbm, e4_hbm, out_hbm, acc, sidx, didx, rows, *sems):
        c = lax.axis_index("c")
        s = lax.axis_index("s")
        wid = c * NS + s
        zero16 = jnp.zeros((16,), jnp.float32)

        def zero(i, carry):
            for u in range(4):
                rows[4 * i + u] = zero16
            return carry

        lax.fori_loop(0, rz // 4, zero, 0)
        pltpu.sync_copy(rows.at[pl.ds(0, rz)], acc.at[pl.ds(s * rz, rz)])
        pltpu.sync_copy(e4_hbm.at[0, pl.ds(wid * ew, ew)], sidx)
        pltpu.sync_copy(e4_hbm.at[1, pl.ds(wid * ew, ew)], didx)
        plsc.subcore_barrier()

        def gfire(k):
            def go(j, carry):
                b = k * cb + j
                pltpu.async_copy(g_hbm.at[sidx.at[pl.ds(b * eb, eb)]],
                                 rows.at[pl.ds(b * eb, eb)], sems[k])
                return carry
            lax.fori_loop(0, cb, go, 0)
            if k == n_chunk - 1 and tail:
                pltpu.async_copy(g_hbm.at[sidx.at[pl.ds(nb * eb, tail)]],
                                 rows.at[pl.ds(nb * eb, tail)], sems[k])

        def sfire(k):
            def go(j, carry):
                b = k * cb + j
                pltpu.async_copy(rows.at[pl.ds(b * eb, eb)],
                                 acc.at[didx.at[pl.ds(b * eb, eb)]],
                                 sems[k], add=True)
                return carry
            lax.fori_loop(0, cb, go, 0)
            if k == n_chunk - 1 and tail:
                pltpu.async_copy(rows.at[pl.ds(nb * eb, tail)],
                                 acc.at[didx.at[pl.ds(nb * eb, tail)]],
                                 sems[k], add=True)

        def drain(k):
            ce = cb * eb + (tail if k == n_chunk - 1 else 0)
            pltpu.make_async_copy(g_hbm.at[pl.ds(0, ce)],
                                  rows.at[pl.ds(0, ce)], sems[k]).wait()

        for k in range(n_chunk):
            gfire(k)
        for k in range(n_chunk):
            drain(k)
            sfire(k)
        for k in range(n_chunk):
            drain(k)
        plsc.subcore_barrier()
        pltpu.sync_copy(acc.at[pl.ds(s * rz, rz)],
                        out_hbm.at[c, pl.ds(s * rz, rz)])

    return prop_kernel


def _make_sc_layer1(r_pad):
    rw = r_pad // NW

    @functools.partial(
        pl.kernel,
        out_type=jax.ShapeDtypeStruct((r_pad, 16), jnp.float32),
        mesh=_mesh(),
        scratch_types=[
            pltpu.VMEM((rw, 16), jnp.float32),
            pltpu.VMEM((rw, 16), jnp.float32),
            pltpu.VMEM((rw, 16), jnp.float32),
            pltpu.VMEM((rw, 16), jnp.float32),
            pltpu.VMEM((16,), jnp.float32),
        ],
        **_SC_PARAMS,
    )
    def l1_kernel(s1_hbm, g0_hbm, dv_hbm, b1_hbm, g1_hbm,
                  abuf, bbuf, gbuf, dbuf, bias):
        c = lax.axis_index("c")
        s = lax.axis_index("s")
        wid = c * NS + s
        sl = pl.ds(wid * rw, rw)
        pltpu.sync_copy(s1_hbm.at[0, sl], abuf)
        pltpu.sync_copy(s1_hbm.at[1, sl], bbuf)
        pltpu.sync_copy(g0_hbm.at[sl], gbuf)
        pltpu.sync_copy(dv_hbm.at[sl], dbuf)
        pltpu.sync_copy(b1_hbm, bias)
        b1v = bias[...]

        def rows(k4, carry):
            for u in range(4):
                k = 4 * k4 + u
                t = abuf[k] + bbuf[k] + gbuf[k]
                h1 = jnp.maximum(dbuf[k] * t + b1v, 0.0)
                gbuf[k] = dbuf[k] * h1
            return carry

        lax.fori_loop(0, rw // 4, rows, 0)
        pltpu.sync_copy(gbuf, g1_hbm.at[sl])

    return l1_kernel


def _make_sc_layer2c(r_pad):
    rw = r_pad // NW

    @functools.partial(
        pl.kernel,
        out_type=jax.ShapeDtypeStruct((r_pad, 128), jnp.float32),
        mesh=_mesh(),
        scratch_types=[
            pltpu.VMEM((rw, 16), jnp.float32),
            pltpu.VMEM((rw, 16), jnp.float32),
            pltpu.VMEM((rw, 16), jnp.float32),
            pltpu.VMEM((rw, 16), jnp.float32),
        ],
        **_SC_PARAMS,
    )
    def l2_kernel(s2_hbm, g1_hbm, dv_hbm, t_hbm, abuf, bbuf, gbuf, dbuf):
        c = lax.axis_index("c")
        s = lax.axis_index("s")
        wid = c * NS + s
        sl = pl.ds(wid * rw, rw)
        pltpu.sync_copy(s2_hbm.at[0, sl], abuf)
        pltpu.sync_copy(s2_hbm.at[1, sl], bbuf)
        pltpu.sync_copy(g1_hbm.at[sl], gbuf)
        pltpu.sync_copy(dv_hbm.at[sl], dbuf)

        def rows(k4, carry):
            for u in range(4):
                k = 4 * k4 + u
                gbuf[k] = dbuf[k] * (abuf[k] + bbuf[k] + gbuf[k])
            return carry

        lax.fori_loop(0, rw // 4, rows, 0)
        pltpu.sync_copy(gbuf, t_hbm.at[sl, pl.ds(0, 16)])

    return l2_kernel


def _tc_matmul(x, wp, r_pad, bn):
    m, k = x.shape

    def body(x_ref, w_ref, o_ref):
        o_ref[...] = jnp.dot(x_ref[...], w_ref[...],
                             preferred_element_type=jnp.float32)

    return pl.pallas_call(
        body,
        grid=(m // bn,),
        in_specs=[
            pl.BlockSpec((bn, k), lambda i: (i, 0)),
            pl.BlockSpec((k, 128), lambda i: (0, 0)),
        ],
        out_specs=pl.BlockSpec((bn, 128), lambda i: (i, 0)),
        out_shape=jax.ShapeDtypeStruct((r_pad, 128), jnp.float32),
    )(x, wp)


def _tc_final(t, w2, b2, n, bn):
    cdim = w2.shape[1]

    def body(t_ref, w2_ref, b2_ref, o_ref):
        t16 = t_ref[:, :16]
        z = jnp.dot(t16, w2_ref[...], preferred_element_type=jnp.float32)
        z = z + b2_ref[...]
        m = jnp.max(z, axis=1, keepdims=True)
        ez = jnp.exp(z - m)
        lse = jnp.log(jnp.sum(ez, axis=1, keepdims=True)) + m
        o_ref[...] = (z - lse).T

    r_pad = t.shape[0]
    return pl.pallas_call(
        body,
        grid=(r_pad // bn,),
        in_specs=[
            pl.BlockSpec((bn, 128), lambda i: (i, 0)),
            pl.BlockSpec((16, cdim), lambda i: (0, 0)),
            pl.BlockSpec((1, cdim), lambda i: (0, 0)),
        ],
        out_specs=pl.BlockSpec((cdim, bn), lambda i: (0, i)),
        out_shape=jax.ShapeDtypeStruct((cdim, n), jnp.float32),
    )(t, w2, b2)


def _make_sc_scale(r_pad):
    rw = r_pad // NW

    @functools.partial(
        pl.kernel,
        out_type=jax.ShapeDtypeStruct((r_pad, 16), jnp.float32),
        mesh=_mesh(),
        scratch_types=[
            pltpu.VMEM((rw, 16), jnp.float32),
            pltpu.VMEM((rw, 16), jnp.float32),
        ],
        **_SC_PARAMS,
    )
    def scale_kernel(dv_hbm, h0_hbm, g0_hbm, dbuf, hbuf):
        c = lax.axis_index("c")
        s = lax.axis_index("s")
        wid = c * NS + s
        pltpu.sync_copy(dv_hbm.at[pl.ds(wid * rw, rw)], dbuf)
        pltpu.sync_copy(h0_hbm.at[pl.ds(wid * rw, rw), pl.ds(0, 16)], hbuf)

        def rows(k4, carry):
            for u in range(4):
                k = 4 * k4 + u
                hbuf[k] = dbuf[k] * hbuf[k]
            return carry

        lax.fori_loop(0, rw // 4, rows, 0)
        pltpu.sync_copy(hbuf, g0_hbm.at[pl.ds(wid * rw, rw)])

    return scale_kernel


def kernel(input_matrix, edge_index, W1, b1, W2, b2):
    n, d = input_matrix.shape
    h = W1.shape[1]
    c = W2.shape[1]
    e = edge_index.shape[1]

    assert e % (NW * 8) == 0

    r_pad = -(-n // (16 * NS)) * (16 * NS)

    bn = 2000

    dv = _make_sc_deg(r_pad, e // NS)(edge_index)
    w1p = jnp.pad(W1, ((0, 0), (0, 128 - h)))
    h0 = _tc_matmul(input_matrix, w1p, r_pad, bn)
    g0 = _make_sc_scale(r_pad)(dv, h0)

    prop = _make_sc_prop(r_pad, e // NW, 8)
    s1 = prop(g0, edge_index)
    g1 = _make_sc_layer1(r_pad)(s1, g0, dv, b1)

    s2 = prop(g1, edge_index)
    t = _make_sc_layer2c(r_pad)(s2, g1, dv)
    return _tc_final(t, W2, b2.reshape(1, c), n, 2048).T
